# baseline (speedup 1.0000x reference)
import functools

import jax
import jax.numpy as jnp
from jax import lax
from jax.experimental import pallas as pl
from jax.experimental.pallas import tpu as pltpu
from jax.experimental.pallas import tpu_sc as plsc

N = 10000
E = 320000
DF = 128
H = 20
C = 16
W = 128

NC, NS = 2, 16
NW = NC * NS
PAD_N = 10240
TRASH = N
CHUNK = 128
CPW = 80
NCH = NW * CPW
E_PAD = NCH * CHUNK
RPT = PAD_N // NS

_mesh = lambda: plsc.VectorSubcoreMesh(core_axis_name="c", subcore_axis_name="s")
_sc_params = lambda: pltpu.CompilerParams(use_tc_tiling_on_sc=False)


@functools.lru_cache(maxsize=None)
def _deg_kernel():
    @functools.partial(
        pl.kernel, mesh=_mesh(), compiler_params=_sc_params(),
        out_type=jax.ShapeDtypeStruct((NC, PAD_N), jnp.float32),
        scratch_types=[
            pltpu.VMEM((CHUNK,), jnp.int32),
            pltpu.VMEM((CHUNK,), jnp.float32),
            pltpu.VMEM_SHARED((PAD_N,), jnp.float32),
            pltpu.SemaphoreType.DMA,
        ],
    )
    def k(dstc_hbm, zeros_hbm, out_hbm, dst_v, ones_v, deg_sh, sem):
        c = lax.axis_index("c")
        s = lax.axis_index("s")
        wid = c * NS + s
        for i in range(CHUNK // 16):
            ones_v[pl.ds(i * 16, 16)] = jnp.ones((16,), jnp.float32)
        r0 = s * RPT
        pltpu.sync_copy(zeros_hbm.at[pl.ds(r0, RPT)], deg_sh.at[pl.ds(r0, RPT)])
        plsc.subcore_barrier()

        def body(j, carry):
            cid = wid * CPW + j
            pltpu.sync_copy(dstc_hbm.at[cid], dst_v)
            pltpu.sync_copy(ones_v, deg_sh.at[dst_v], add=True)
            return carry

        lax.fori_loop(0, CPW, body, 0)
        plsc.subcore_barrier()
        pltpu.sync_copy(deg_sh.at[pl.ds(r0, RPT)], out_hbm.at[c].at[pl.ds(r0, RPT)])

    return k


@functools.lru_cache(maxsize=None)
def _agg_kernel():
    @functools.partial(
        pl.kernel, mesh=_mesh(), compiler_params=_sc_params(),
        out_type=jax.ShapeDtypeStruct((NC, PAD_N, W), jnp.float32),
        scratch_types=[
            pltpu.VMEM((2, CHUNK), jnp.int32),
            pltpu.VMEM((2, CHUNK), jnp.int32),
            pltpu.VMEM((CHUNK, W), jnp.float32),
            pltpu.VMEM((CHUNK, W), jnp.float32),
            pltpu.VMEM_SHARED((PAD_N, W), jnp.float32),
            pltpu.SemaphoreType.DMA,
            pltpu.SemaphoreType.DMA,
        ],
    )
    def k(hp_hbm, eidx_hbm, zeros_hbm, out_hbm,
          ev0, ev1, rows0, rows1, agg_sh, sem0, sem1):
        c = lax.axis_index("c")
        s = lax.axis_index("s")
        wid = c * NS + s
        base = wid * CPW
        r0 = s * RPT
        ev = (ev0, ev1)
        rows = (rows0, rows1)
        sem = (sem0, sem1)
        pltpu.sync_copy(zeros_hbm.at[pl.ds(r0, RPT)], agg_sh.at[pl.ds(r0, RPT)])
        plsc.subcore_barrier()

        pltpu.sync_copy(eidx_hbm.at[base], ev0)
        pltpu.async_copy(hp_hbm.at[ev0.at[0]], rows0, sem0)

        def body(i, carry):
            j = i * 2
            for b in range(2):
                cid = j + b
                nxt = cid + 1

                @pl.when(nxt < CPW)
                def _():
                    pltpu.sync_copy(eidx_hbm.at[base + nxt], ev[1 - b])
                    pltpu.async_copy(hp_hbm.at[ev[1 - b].at[0]],
                                     rows[1 - b], sem[1 - b])

                pltpu.make_async_copy(hp_hbm.at[ev[b].at[0]],
                                      rows[b], sem[b]).wait()
                pltpu.sync_copy(rows[b], agg_sh.at[ev[b].at[1]], add=True)
            return carry

        lax.fori_loop(0, CPW // 2, body, 0)
        plsc.subcore_barrier()
        pltpu.sync_copy(agg_sh.at[pl.ds(r0, RPT)],
                        out_hbm.at[c].at[pl.ds(r0, RPT)])

    return k


def _tc_scale_matmul(x_p, W1, degt):
    BN = 1024

    def body(x_ref, w_ref, deg_ref, hp_ref, dinv_ref):
        deg = deg_ref[:, 0:1] + deg_ref[:, 1:2] + 1.0
        dinv = lax.rsqrt(deg)
        h = jnp.dot(x_ref[...], w_ref[...], preferred_element_type=jnp.float32)
        hp_ref[...] = jnp.pad(h * dinv, ((0, 0), (0, W - H)))
        dinv_ref[...] = dinv

    return pl.pallas_call(
        body,
        grid=(PAD_N // BN,),
        in_specs=[
            pl.BlockSpec((BN, DF), lambda i: (i, 0)),
            pl.BlockSpec((DF, H), lambda i: (0, 0)),
            pl.BlockSpec((BN, NC), lambda i: (i, 0)),
        ],
        out_specs=[
            pl.BlockSpec((BN, W), lambda i: (i, 0)),
            pl.BlockSpec((BN, 1), lambda i: (i, 0)),
        ],
        out_shape=[
            jax.ShapeDtypeStruct((PAD_N, W), jnp.float32),
            jax.ShapeDtypeStruct((PAD_N, 1), jnp.float32),
        ],
    )(x_p, W1, degt)


def _tc_mid(aggp, hp, dinv, b1, W2):
    BN = 1024

    def body(a_ref, hp_ref, dinv_ref, b1_ref, w2_ref, out_ref):
        p = (a_ref[0] + a_ref[1] + hp_ref[...])[:, :H]
        h1 = jnp.maximum(dinv_ref[...] * p + b1_ref[...], 0.0)
        h2 = jnp.dot(h1, w2_ref[...], preferred_element_type=jnp.float32)
        out_ref[...] = jnp.pad(h2 * dinv_ref[...], ((0, 0), (0, W - C)))

    return pl.pallas_call(
        body,
        grid=(PAD_N // BN,),
        in_specs=[
            pl.BlockSpec((NC, BN, W), lambda i: (0, i, 0)),
            pl.BlockSpec((BN, W), lambda i: (i, 0)),
            pl.BlockSpec((BN, 1), lambda i: (i, 0)),
            pl.BlockSpec((1, H), lambda i: (0, 0)),
            pl.BlockSpec((H, C), lambda i: (0, 0)),
        ],
        out_specs=pl.BlockSpec((BN, W), lambda i: (i, 0)),
        out_shape=jax.ShapeDtypeStruct((PAD_N, W), jnp.float32),
    )(aggp, hp, dinv, b1, W2)


def _tc_final(aggp, h2p, dinv, b2):
    BN = 1024

    def body(a_ref, hp_ref, dinv_ref, b2_ref, out_ref):
        q = (a_ref[0] + a_ref[1] + hp_ref[...])[:, :C]
        z = dinv_ref[...] * q + b2_ref[...]
        m = jnp.max(z, axis=1, keepdims=True)
        e = jnp.exp(z - m)
        out_ref[...] = (z - m) - jnp.log(jnp.sum(e, axis=1, keepdims=True))

    return pl.pallas_call(
        body,
        grid=(PAD_N // BN,),
        in_specs=[
            pl.BlockSpec((NC, BN, W), lambda i: (0, i, 0)),
            pl.BlockSpec((BN, W), lambda i: (i, 0)),
            pl.BlockSpec((BN, 1), lambda i: (i, 0)),
            pl.BlockSpec((1, C), lambda i: (0, 0)),
        ],
        out_specs=pl.BlockSpec((BN, C), lambda i: (i, 0)),
        out_shape=jax.ShapeDtypeStruct((PAD_N, C), jnp.float32),
    )(aggp, h2p, dinv, b2)


def kernel(x, edge_index, W1, b1, W2, b2):
    ei = edge_index.astype(jnp.int32)
    pad_e = E_PAD - E
    srcc = jnp.concatenate(
        [ei[0], jnp.zeros((pad_e,), jnp.int32)]).reshape(NCH, 1, CHUNK)
    dstc = jnp.concatenate(
        [ei[1], jnp.full((pad_e,), TRASH, jnp.int32)]).reshape(NCH, 1, CHUNK)
    eidx = jnp.concatenate([srcc, dstc], axis=1)
    x_p = jnp.pad(x, ((0, PAD_N - N), (0, 0)))
    z1 = jnp.zeros((PAD_N,), jnp.float32)
    zw = jnp.zeros((PAD_N, W), jnp.float32)

    degp = _deg_kernel()(dstc.reshape(NCH, CHUNK), z1)
    degt = degp.T
    hp, dinv = _tc_scale_matmul(x_p, W1, degt)
    agg1 = _agg_kernel()(hp, eidx, zw)
    h2p = _tc_mid(agg1, hp, dinv, b1.reshape(1, H), W2)
    agg2 = _agg_kernel()(h2p, eidx, zw)
    out = _tc_final(agg2, h2p, dinv, b2.reshape(1, C))
    return out[:N]


# speedup vs baseline: 3.4954x; 3.4954x over previous
import functools

import jax
import jax.numpy as jnp
from jax import lax
from jax.experimental import pallas as pl
from jax.experimental.pallas import tpu as pltpu
from jax.experimental.pallas import tpu_sc as plsc

N = 10000
E = 320000
DF = 128
H = 20
C = 16
W = 128

NC, NS = 2, 16
NW = NC * NS
PAD_N = 10240
TRASH = N
CHUNK = 128
CPW = 80
NCH = NW * CPW
E_PAD = NCH * CHUNK
RPT = PAD_N // NS

_mesh = lambda: plsc.VectorSubcoreMesh(core_axis_name="c", subcore_axis_name="s")
_sc_params = lambda: pltpu.CompilerParams(use_tc_tiling_on_sc=False)


@functools.lru_cache(maxsize=None)
def _deg_kernel():
    @functools.partial(
        pl.kernel, mesh=_mesh(), compiler_params=_sc_params(),
        out_type=jax.ShapeDtypeStruct((NC, PAD_N), jnp.float32),
        scratch_types=[
            pltpu.VMEM((CHUNK,), jnp.int32),
            pltpu.VMEM((CHUNK,), jnp.float32),
            pltpu.VMEM_SHARED((PAD_N,), jnp.float32),
            pltpu.SemaphoreType.DMA,
        ],
    )
    def k(dstc_hbm, zeros_hbm, out_hbm, dst_v, ones_v, deg_sh, sem):
        c = lax.axis_index("c")
        s = lax.axis_index("s")
        wid = c * NS + s
        for i in range(CHUNK // 16):
            ones_v[pl.ds(i * 16, 16)] = jnp.ones((16,), jnp.float32)
        r0 = s * RPT
        pltpu.sync_copy(zeros_hbm.at[pl.ds(r0, RPT)], deg_sh.at[pl.ds(r0, RPT)])
        plsc.subcore_barrier()

        def body(j, carry):
            cid = wid * CPW + j
            pltpu.sync_copy(dstc_hbm.at[cid], dst_v)
            pltpu.sync_copy(ones_v, deg_sh.at[dst_v], add=True)
            return carry

        lax.fori_loop(0, CPW, body, 0)
        plsc.subcore_barrier()
        pltpu.sync_copy(deg_sh.at[pl.ds(r0, RPT)], out_hbm.at[c].at[pl.ds(r0, RPT)])

    return k


WS = 32


@functools.lru_cache(maxsize=None)
def _agg_kernel():
    @functools.partial(
        pl.kernel, mesh=_mesh(), compiler_params=_sc_params(),
        out_type=jax.ShapeDtypeStruct((NC, PAD_N, W), jnp.float32),
        scratch_types=[
            pltpu.VMEM((2, CHUNK), jnp.int32),
            pltpu.VMEM((CHUNK, WS), jnp.float32),
            pltpu.VMEM_SHARED((PAD_N, WS), jnp.float32),
            pltpu.VMEM_SHARED((PAD_N, WS), jnp.float32),
            pltpu.SemaphoreType.DMA,
        ],
    )
    def k(hp_hbm, eidx_hbm, zeros_hbm, out_hbm,
          ev, rows_v, hp_sh, agg_sh, sem):
        c = lax.axis_index("c")
        s = lax.axis_index("s")
        wid = c * NS + s
        base = wid * CPW
        r0 = s * RPT
        pltpu.sync_copy(zeros_hbm.at[pl.ds(r0, RPT)], agg_sh.at[pl.ds(r0, RPT)])
        pltpu.sync_copy(hp_hbm.at[pl.ds(r0, RPT), pl.ds(0, WS)],
                        hp_sh.at[pl.ds(r0, RPT)])
        plsc.subcore_barrier()

        def body(j, carry):
            cid = base + j
            pltpu.sync_copy(eidx_hbm.at[cid], ev)
            pltpu.async_copy(hp_sh.at[ev.at[0]], rows_v, sem).wait()
            pltpu.sync_copy(rows_v, agg_sh.at[ev.at[1]], add=True)
            return carry

        lax.fori_loop(0, CPW, body, 0)
        plsc.subcore_barrier()
        pltpu.sync_copy(agg_sh.at[pl.ds(r0, RPT)],
                        out_hbm.at[c].at[pl.ds(r0, RPT), pl.ds(0, WS)])

    return k


def _tc_scale_matmul(x_p, W1, degt):
    BN = 1024

    def body(x_ref, w_ref, deg_ref, hp_ref, dinv_ref):
        deg = deg_ref[:, 0:1] + deg_ref[:, 1:2] + 1.0
        dinv = lax.rsqrt(deg)
        h = jnp.dot(x_ref[...], w_ref[...], preferred_element_type=jnp.float32)
        hp_ref[...] = jnp.pad(h * dinv, ((0, 0), (0, W - H)))
        dinv_ref[...] = dinv

    return pl.pallas_call(
        body,
        grid=(PAD_N // BN,),
        in_specs=[
            pl.BlockSpec((BN, DF), lambda i: (i, 0)),
            pl.BlockSpec((DF, H), lambda i: (0, 0)),
            pl.BlockSpec((BN, NC), lambda i: (i, 0)),
        ],
        out_specs=[
            pl.BlockSpec((BN, W), lambda i: (i, 0)),
            pl.BlockSpec((BN, 1), lambda i: (i, 0)),
        ],
        out_shape=[
            jax.ShapeDtypeStruct((PAD_N, W), jnp.float32),
            jax.ShapeDtypeStruct((PAD_N, 1), jnp.float32),
        ],
    )(x_p, W1, degt)


def _tc_mid(aggp, hp, dinv, b1, W2):
    BN = 1024

    def body(a_ref, hp_ref, dinv_ref, b1_ref, w2_ref, out_ref):
        p = (a_ref[0] + a_ref[1] + hp_ref[...])[:, :H]
        h1 = jnp.maximum(dinv_ref[...] * p + b1_ref[...], 0.0)
        h2 = jnp.dot(h1, w2_ref[...], preferred_element_type=jnp.float32)
        out_ref[...] = jnp.pad(h2 * dinv_ref[...], ((0, 0), (0, W - C)))

    return pl.pallas_call(
        body,
        grid=(PAD_N // BN,),
        in_specs=[
            pl.BlockSpec((NC, BN, W), lambda i: (0, i, 0)),
            pl.BlockSpec((BN, W), lambda i: (i, 0)),
            pl.BlockSpec((BN, 1), lambda i: (i, 0)),
            pl.BlockSpec((1, H), lambda i: (0, 0)),
            pl.BlockSpec((H, C), lambda i: (0, 0)),
        ],
        out_specs=pl.BlockSpec((BN, W), lambda i: (i, 0)),
        out_shape=jax.ShapeDtypeStruct((PAD_N, W), jnp.float32),
    )(aggp, hp, dinv, b1, W2)


def _tc_final(aggp, h2p, dinv, b2):
    BN = 1024

    def body(a_ref, hp_ref, dinv_ref, b2_ref, out_ref):
        q = (a_ref[0] + a_ref[1] + hp_ref[...])[:, :C]
        z = dinv_ref[...] * q + b2_ref[...]
        m = jnp.max(z, axis=1, keepdims=True)
        e = jnp.exp(z - m)
        out_ref[...] = (z - m) - jnp.log(jnp.sum(e, axis=1, keepdims=True))

    return pl.pallas_call(
        body,
        grid=(PAD_N // BN,),
        in_specs=[
            pl.BlockSpec((NC, BN, W), lambda i: (0, i, 0)),
            pl.BlockSpec((BN, W), lambda i: (i, 0)),
            pl.BlockSpec((BN, 1), lambda i: (i, 0)),
            pl.BlockSpec((1, C), lambda i: (0, 0)),
        ],
        out_specs=pl.BlockSpec((BN, C), lambda i: (i, 0)),
        out_shape=jax.ShapeDtypeStruct((PAD_N, C), jnp.float32),
    )(aggp, h2p, dinv, b2)


def kernel(x, edge_index, W1, b1, W2, b2):
    ei = edge_index.astype(jnp.int32)
    pad_e = E_PAD - E
    srcc = jnp.concatenate(
        [ei[0], jnp.zeros((pad_e,), jnp.int32)]).reshape(NCH, 1, CHUNK)
    dstc = jnp.concatenate(
        [ei[1], jnp.full((pad_e,), TRASH, jnp.int32)]).reshape(NCH, 1, CHUNK)
    eidx = jnp.concatenate([srcc, dstc], axis=1)
    x_p = jnp.pad(x, ((0, PAD_N - N), (0, 0)))
    z1 = jnp.zeros((PAD_N,), jnp.float32)
    zw = jnp.zeros((PAD_N, WS), jnp.float32)

    degp = _deg_kernel()(dstc.reshape(NCH, CHUNK), z1)
    degt = degp.T
    hp, dinv = _tc_scale_matmul(x_p, W1, degt)
    agg1 = _agg_kernel()(hp, eidx, zw)
    h2p = _tc_mid(agg1, hp, dinv, b1.reshape(1, H), W2)
    agg2 = _agg_kernel()(h2p, eidx, zw)
    out = _tc_final(agg2, h2p, dinv, b2.reshape(1, C))
    return out[:N]


# speedup vs baseline: 5.9475x; 1.7015x over previous
import functools

import jax
import jax.numpy as jnp
from jax import lax
from jax.experimental import pallas as pl
from jax.experimental.pallas import tpu as pltpu
from jax.experimental.pallas import tpu_sc as plsc

N = 10000
E = 320000
DF = 128
H = 20
C = 16
W = 128

NC, NS = 2, 16
NW = NC * NS
PAD_N = 10240
TRASH = N
CHUNK = 128
CPW = 80
NCH = NW * CPW
E_PAD = NCH * CHUNK
RPT = PAD_N // NS

_mesh = lambda: plsc.VectorSubcoreMesh(core_axis_name="c", subcore_axis_name="s")
_sc_params = lambda: pltpu.CompilerParams(use_tc_tiling_on_sc=False)


@functools.lru_cache(maxsize=None)
def _deg_kernel():
    @functools.partial(
        pl.kernel, mesh=_mesh(), compiler_params=_sc_params(),
        out_type=jax.ShapeDtypeStruct((NC, PAD_N), jnp.float32),
        scratch_types=[
            pltpu.VMEM((CPW, CHUNK), jnp.int32),
            pltpu.VMEM((CHUNK,), jnp.float32),
            pltpu.VMEM_SHARED((PAD_N,), jnp.float32),
            pltpu.SemaphoreType.DMA,
        ],
    )
    def k(dstc_hbm, zeros_hbm, out_hbm, dst_v, ones_v, deg_sh, sem):
        c = lax.axis_index("c")
        s = lax.axis_index("s")
        wid = c * NS + s
        for i in range(CHUNK // 16):
            ones_v[pl.ds(i * 16, 16)] = jnp.ones((16,), jnp.float32)
        r0 = s * RPT
        pltpu.sync_copy(zeros_hbm.at[pl.ds(r0, RPT)], deg_sh.at[pl.ds(r0, RPT)])
        pltpu.sync_copy(dstc_hbm.at[pl.ds(wid * CPW, CPW)], dst_v)
        plsc.subcore_barrier()

        def body(j, carry):
            pltpu.sync_copy(ones_v, deg_sh.at[dst_v.at[j]], add=True)
            return carry

        lax.fori_loop(0, CPW, body, 0)
        plsc.subcore_barrier()
        pltpu.sync_copy(deg_sh.at[pl.ds(r0, RPT)], out_hbm.at[c].at[pl.ds(r0, RPT)])

    return k


@functools.lru_cache(maxsize=None)
def _agg_kernel(ws):
    @functools.partial(
        pl.kernel, mesh=_mesh(), compiler_params=_sc_params(),
        out_type=jax.ShapeDtypeStruct((NC, PAD_N, W), jnp.float32),
        scratch_types=[
            pltpu.VMEM((CPW, 2, CHUNK), jnp.int32),
            pltpu.VMEM((CHUNK, ws), jnp.float32),
            pltpu.VMEM_SHARED((PAD_N, ws), jnp.float32),
            pltpu.VMEM_SHARED((PAD_N, ws), jnp.float32),
            pltpu.SemaphoreType.DMA,
        ],
    )
    def k(hp_hbm, eidx_hbm, zeros_hbm, out_hbm,
          ev, rows_v, hp_sh, agg_sh, sem):
        c = lax.axis_index("c")
        s = lax.axis_index("s")
        wid = c * NS + s
        r0 = s * RPT
        pltpu.sync_copy(zeros_hbm.at[pl.ds(r0, RPT)], agg_sh.at[pl.ds(r0, RPT)])
        pltpu.sync_copy(hp_hbm.at[pl.ds(r0, RPT), pl.ds(0, ws)],
                        hp_sh.at[pl.ds(r0, RPT)])
        pltpu.sync_copy(eidx_hbm.at[pl.ds(wid * CPW, CPW)], ev)
        plsc.subcore_barrier()

        def body(j, carry):
            pltpu.async_copy(hp_sh.at[ev.at[j].at[0]], rows_v, sem).wait()
            pltpu.sync_copy(rows_v, agg_sh.at[ev.at[j].at[1]], add=True)
            return carry

        lax.fori_loop(0, CPW, body, 0)
        plsc.subcore_barrier()
        pltpu.sync_copy(agg_sh.at[pl.ds(r0, RPT)],
                        out_hbm.at[c].at[pl.ds(r0, RPT), pl.ds(0, ws)])

    return k


def _tc_scale_matmul(x_p, W1, degt):
    BN = 1024

    def body(x_ref, w_ref, deg_ref, hp_ref, dinv_ref):
        deg = deg_ref[:, 0:1] + deg_ref[:, 1:2] + 1.0
        dinv = lax.rsqrt(deg)
        h = jnp.dot(x_ref[...], w_ref[...], preferred_element_type=jnp.float32)
        hp_ref[...] = jnp.pad(h * dinv, ((0, 0), (0, W - H)))
        dinv_ref[...] = dinv

    return pl.pallas_call(
        body,
        grid=(PAD_N // BN,),
        in_specs=[
            pl.BlockSpec((BN, DF), lambda i: (i, 0)),
            pl.BlockSpec((DF, H), lambda i: (0, 0)),
            pl.BlockSpec((BN, NC), lambda i: (i, 0)),
        ],
        out_specs=[
            pl.BlockSpec((BN, W), lambda i: (i, 0)),
            pl.BlockSpec((BN, 1), lambda i: (i, 0)),
        ],
        out_shape=[
            jax.ShapeDtypeStruct((PAD_N, W), jnp.float32),
            jax.ShapeDtypeStruct((PAD_N, 1), jnp.float32),
        ],
    )(x_p, W1, degt)


def _tc_mid(aggp, hp, dinv, b1, W2):
    BN = 1024

    def body(a_ref, hp_ref, dinv_ref, b1_ref, w2_ref, out_ref):
        p = (a_ref[0] + a_ref[1] + hp_ref[...])[:, :H]
        h1 = jnp.maximum(dinv_ref[...] * p + b1_ref[...], 0.0)
        h2 = jnp.dot(h1, w2_ref[...], preferred_element_type=jnp.float32)
        out_ref[...] = jnp.pad(h2 * dinv_ref[...], ((0, 0), (0, W - C)))

    return pl.pallas_call(
        body,
        grid=(PAD_N // BN,),
        in_specs=[
            pl.BlockSpec((NC, BN, W), lambda i: (0, i, 0)),
            pl.BlockSpec((BN, W), lambda i: (i, 0)),
            pl.BlockSpec((BN, 1), lambda i: (i, 0)),
            pl.BlockSpec((1, H), lambda i: (0, 0)),
            pl.BlockSpec((H, C), lambda i: (0, 0)),
        ],
        out_specs=pl.BlockSpec((BN, W), lambda i: (i, 0)),
        out_shape=jax.ShapeDtypeStruct((PAD_N, W), jnp.float32),
    )(aggp, hp, dinv, b1, W2)


def _tc_final(aggp, h2p, dinv, b2):
    BN = 1024

    def body(a_ref, hp_ref, dinv_ref, b2_ref, out_ref):
        q = (a_ref[0] + a_ref[1] + hp_ref[...])[:, :C]
        z = dinv_ref[...] * q + b2_ref[...]
        m = jnp.max(z, axis=1, keepdims=True)
        e = jnp.exp(z - m)
        out_ref[...] = (z - m) - jnp.log(jnp.sum(e, axis=1, keepdims=True))

    return pl.pallas_call(
        body,
        grid=(PAD_N // BN,),
        in_specs=[
            pl.BlockSpec((NC, BN, W), lambda i: (0, i, 0)),
            pl.BlockSpec((BN, W), lambda i: (i, 0)),
            pl.BlockSpec((BN, 1), lambda i: (i, 0)),
            pl.BlockSpec((1, C), lambda i: (0, 0)),
        ],
        out_specs=pl.BlockSpec((BN, C), lambda i: (i, 0)),
        out_shape=jax.ShapeDtypeStruct((PAD_N, C), jnp.float32),
    )(aggp, h2p, dinv, b2)


def kernel(x, edge_index, W1, b1, W2, b2):
    ei = edge_index.astype(jnp.int32)
    pad_e = E_PAD - E
    srcc = jnp.concatenate(
        [ei[0], jnp.zeros((pad_e,), jnp.int32)]).reshape(NCH, 1, CHUNK)
    dstc = jnp.concatenate(
        [ei[1], jnp.full((pad_e,), TRASH, jnp.int32)]).reshape(NCH, 1, CHUNK)
    eidx = jnp.concatenate([srcc, dstc], axis=1)
    x_p = jnp.pad(x, ((0, PAD_N - N), (0, 0)))
    z1 = jnp.zeros((PAD_N,), jnp.float32)
    za = jnp.zeros((PAD_N, 24), jnp.float32)
    zb = jnp.zeros((PAD_N, 16), jnp.float32)

    degp = _deg_kernel()(dstc.reshape(NCH, CHUNK), z1)
    degt = degp.T
    hp, dinv = _tc_scale_matmul(x_p, W1, degt)
    agg1 = _agg_kernel(24)(hp, eidx, za)
    h2p = _tc_mid(agg1, hp, dinv, b1.reshape(1, H), W2)
    agg2 = _agg_kernel(16)(h2p, eidx, zb)
    out = _tc_final(agg2, h2p, dinv, b2.reshape(1, C))
    return out[:N]


# speedup vs baseline: 6.7011x; 1.1267x over previous
import functools

import jax
import jax.numpy as jnp
from jax import lax
from jax.experimental import pallas as pl
from jax.experimental.pallas import tpu as pltpu
from jax.experimental.pallas import tpu_sc as plsc

N = 10000
E = 320000
DF = 128
H = 20
C = 16
W = 128

NC, NS = 2, 16
NW = NC * NS
PAD_N = 10240
TRASH = N
CHUNK = 128
CPW = 80
NCH = NW * CPW
E_PAD = NCH * CHUNK
RPT = PAD_N // NS

_mesh = lambda: plsc.VectorSubcoreMesh(core_axis_name="c", subcore_axis_name="s")
_sc_params = lambda: pltpu.CompilerParams(use_tc_tiling_on_sc=False)


@functools.lru_cache(maxsize=None)
def _deg_kernel():
    @functools.partial(
        pl.kernel, mesh=_mesh(), compiler_params=_sc_params(),
        out_type=jax.ShapeDtypeStruct((NC, PAD_N), jnp.float32),
        scratch_types=[
            pltpu.VMEM((CPW, CHUNK), jnp.int32),
            pltpu.VMEM((CHUNK,), jnp.float32),
            pltpu.VMEM_SHARED((PAD_N,), jnp.float32),
            pltpu.SemaphoreType.DMA,
        ],
    )
    def k(dstc_hbm, zeros_hbm, out_hbm, dst_v, ones_v, deg_sh, sem):
        c = lax.axis_index("c")
        s = lax.axis_index("s")
        wid = c * NS + s
        for i in range(CHUNK // 16):
            ones_v[pl.ds(i * 16, 16)] = jnp.ones((16,), jnp.float32)
        r0 = s * RPT
        pltpu.sync_copy(zeros_hbm.at[pl.ds(r0, RPT)], deg_sh.at[pl.ds(r0, RPT)])
        pltpu.sync_copy(dstc_hbm.at[pl.ds(wid * CPW, CPW)], dst_v)
        plsc.subcore_barrier()

        def body(j, carry):
            pltpu.sync_copy(ones_v, deg_sh.at[dst_v.at[j]], add=True)
            return carry

        lax.fori_loop(0, CPW, body, 0)
        plsc.subcore_barrier()
        pltpu.sync_copy(deg_sh.at[pl.ds(r0, RPT)], out_hbm.at[c].at[pl.ds(r0, RPT)])

    return k


@functools.lru_cache(maxsize=None)
def _agg_kernel(ws):
    @functools.partial(
        pl.kernel, mesh=_mesh(), compiler_params=_sc_params(),
        out_type=jax.ShapeDtypeStruct((NC, PAD_N, W), jnp.float32),
        scratch_types=[
            pltpu.VMEM((CPW, 2, CHUNK), jnp.int32),
            pltpu.VMEM((CHUNK, ws), jnp.float32),
            pltpu.VMEM((CHUNK, ws), jnp.float32),
            pltpu.VMEM_SHARED((PAD_N, ws), jnp.float32),
            pltpu.VMEM_SHARED((PAD_N, ws), jnp.float32),
            pltpu.SemaphoreType.DMA,
            pltpu.SemaphoreType.DMA,
        ],
    )
    def k(hp_hbm, eidx_hbm, zeros_hbm, out_hbm,
          ev, rows0, rows1, hp_sh, agg_sh, sem0, sem1):
        c = lax.axis_index("c")
        s = lax.axis_index("s")
        wid = c * NS + s
        r0 = s * RPT
        rows = (rows0, rows1)
        sem = (sem0, sem1)
        pltpu.sync_copy(zeros_hbm.at[pl.ds(r0, RPT)], agg_sh.at[pl.ds(r0, RPT)])
        pltpu.sync_copy(hp_hbm.at[pl.ds(r0, RPT), pl.ds(0, ws)],
                        hp_sh.at[pl.ds(r0, RPT)])
        pltpu.sync_copy(eidx_hbm.at[pl.ds(wid * CPW, CPW)], ev)
        plsc.subcore_barrier()

        pltpu.async_copy(hp_sh.at[ev.at[0].at[0]], rows0, sem0)

        def body(i, carry):
            j = i * 2
            for b in range(2):
                cid = j + b
                nxt = cid + 1

                @pl.when(nxt < CPW)
                def _():
                    pltpu.async_copy(hp_sh.at[ev.at[nxt].at[0]],
                                     rows[1 - b], sem[1 - b])

                pltpu.make_async_copy(hp_sh.at[ev.at[cid].at[0]],
                                      rows[b], sem[b]).wait()
                pltpu.sync_copy(rows[b], agg_sh.at[ev.at[cid].at[1]], add=True)
            return carry

        lax.fori_loop(0, CPW // 2, body, 0)
        plsc.subcore_barrier()
        pltpu.sync_copy(agg_sh.at[pl.ds(r0, RPT)],
                        out_hbm.at[c].at[pl.ds(r0, RPT), pl.ds(0, ws)])

    return k


def _tc_scale_matmul(x_p, W1, degt):
    BN = 1024

    def body(x_ref, w_ref, deg_ref, hp_ref, dinv_ref):
        deg = deg_ref[:, 0:1] + deg_ref[:, 1:2] + 1.0
        dinv = lax.rsqrt(deg)
        h = jnp.dot(x_ref[...], w_ref[...], preferred_element_type=jnp.float32)
        hp_ref[...] = jnp.pad(h * dinv, ((0, 0), (0, W - H)))
        dinv_ref[...] = dinv

    return pl.pallas_call(
        body,
        grid=(PAD_N // BN,),
        in_specs=[
            pl.BlockSpec((BN, DF), lambda i: (i, 0)),
            pl.BlockSpec((DF, H), lambda i: (0, 0)),
            pl.BlockSpec((BN, NC), lambda i: (i, 0)),
        ],
        out_specs=[
            pl.BlockSpec((BN, W), lambda i: (i, 0)),
            pl.BlockSpec((BN, 1), lambda i: (i, 0)),
        ],
        out_shape=[
            jax.ShapeDtypeStruct((PAD_N, W), jnp.float32),
            jax.ShapeDtypeStruct((PAD_N, 1), jnp.float32),
        ],
    )(x_p, W1, degt)


def _tc_mid(aggp, hp, dinv, b1, W2):
    BN = 1024

    def body(a_ref, hp_ref, dinv_ref, b1_ref, w2_ref, out_ref):
        p = (a_ref[0] + a_ref[1] + hp_ref[...])[:, :H]
        h1 = jnp.maximum(dinv_ref[...] * p + b1_ref[...], 0.0)
        h2 = jnp.dot(h1, w2_ref[...], preferred_element_type=jnp.float32)
        out_ref[...] = jnp.pad(h2 * dinv_ref[...], ((0, 0), (0, W - C)))

    return pl.pallas_call(
        body,
        grid=(PAD_N // BN,),
        in_specs=[
            pl.BlockSpec((NC, BN, W), lambda i: (0, i, 0)),
            pl.BlockSpec((BN, W), lambda i: (i, 0)),
            pl.BlockSpec((BN, 1), lambda i: (i, 0)),
            pl.BlockSpec((1, H), lambda i: (0, 0)),
            pl.BlockSpec((H, C), lambda i: (0, 0)),
        ],
        out_specs=pl.BlockSpec((BN, W), lambda i: (i, 0)),
        out_shape=jax.ShapeDtypeStruct((PAD_N, W), jnp.float32),
    )(aggp, hp, dinv, b1, W2)


def _tc_final(aggp, h2p, dinv, b2):
    BN = 1024

    def body(a_ref, hp_ref, dinv_ref, b2_ref, out_ref):
        q = (a_ref[0] + a_ref[1] + hp_ref[...])[:, :C]
        z = dinv_ref[...] * q + b2_ref[...]
        m = jnp.max(z, axis=1, keepdims=True)
        e = jnp.exp(z - m)
        out_ref[...] = (z - m) - jnp.log(jnp.sum(e, axis=1, keepdims=True))

    return pl.pallas_call(
        body,
        grid=(PAD_N // BN,),
        in_specs=[
            pl.BlockSpec((NC, BN, W), lambda i: (0, i, 0)),
            pl.BlockSpec((BN, W), lambda i: (i, 0)),
            pl.BlockSpec((BN, 1), lambda i: (i, 0)),
            pl.BlockSpec((1, C), lambda i: (0, 0)),
        ],
        out_specs=pl.BlockSpec((BN, C), lambda i: (i, 0)),
        out_shape=jax.ShapeDtypeStruct((PAD_N, C), jnp.float32),
    )(aggp, h2p, dinv, b2)


def kernel(x, edge_index, W1, b1, W2, b2):
    ei = edge_index.astype(jnp.int32)
    pad_e = E_PAD - E
    srcc = jnp.concatenate(
        [ei[0], jnp.zeros((pad_e,), jnp.int32)]).reshape(NCH, 1, CHUNK)
    dstc = jnp.concatenate(
        [ei[1], jnp.full((pad_e,), TRASH, jnp.int32)]).reshape(NCH, 1, CHUNK)
    eidx = jnp.concatenate([srcc, dstc], axis=1)
    x_p = jnp.pad(x, ((0, PAD_N - N), (0, 0)))
    z1 = jnp.zeros((PAD_N,), jnp.float32)
    za = jnp.zeros((PAD_N, 24), jnp.float32)
    zb = jnp.zeros((PAD_N, 16), jnp.float32)

    degp = _deg_kernel()(dstc.reshape(NCH, CHUNK), z1)
    degt = degp.T
    hp, dinv = _tc_scale_matmul(x_p, W1, degt)
    agg1 = _agg_kernel(24)(hp, eidx, za)
    h2p = _tc_mid(agg1, hp, dinv, b1.reshape(1, H), W2)
    agg2 = _agg_kernel(16)(h2p, eidx, zb)
    out = _tc_final(agg2, h2p, dinv, b2.reshape(1, C))
    return out[:N]


# speedup vs baseline: 6.9526x; 1.0375x over previous
import functools

import jax
import jax.numpy as jnp
from jax import lax
from jax.experimental import pallas as pl
from jax.experimental.pallas import tpu as pltpu
from jax.experimental.pallas import tpu_sc as plsc

N = 10000
E = 320000
DF = 128
H = 20
C = 16
W = 128

NC, NS = 2, 16
NW = NC * NS
PAD_N = 10240
TRASH = N
CHUNK = 128
CPW = 80
NCH = NW * CPW
E_PAD = NCH * CHUNK
RPT = PAD_N // NS

_mesh = lambda: plsc.VectorSubcoreMesh(core_axis_name="c", subcore_axis_name="s")
_sc_params = lambda: pltpu.CompilerParams(use_tc_tiling_on_sc=False, skip_device_barrier=True)


@functools.lru_cache(maxsize=None)
def _deg_kernel():
    @functools.partial(
        pl.kernel, mesh=_mesh(), compiler_params=_sc_params(),
        out_type=jax.ShapeDtypeStruct((NC, PAD_N), jnp.float32),
        scratch_types=[
            pltpu.VMEM((CPW, CHUNK), jnp.int32),
            pltpu.VMEM((CHUNK,), jnp.float32),
            pltpu.VMEM_SHARED((PAD_N,), jnp.float32),
            pltpu.SemaphoreType.DMA,
        ],
    )
    def k(dstc_hbm, zeros_hbm, out_hbm, dst_v, ones_v, deg_sh, sem):
        c = lax.axis_index("c")
        s = lax.axis_index("s")
        wid = c * NS + s
        for i in range(CHUNK // 16):
            ones_v[pl.ds(i * 16, 16)] = jnp.ones((16,), jnp.float32)
        r0 = s * RPT
        pltpu.sync_copy(zeros_hbm.at[pl.ds(r0, RPT)], deg_sh.at[pl.ds(r0, RPT)])
        pltpu.sync_copy(dstc_hbm.at[pl.ds(wid * CPW, CPW)], dst_v)
        plsc.subcore_barrier()

        def body(j, carry):
            pltpu.sync_copy(ones_v, deg_sh.at[dst_v.at[j]], add=True)
            return carry

        lax.fori_loop(0, CPW, body, 0)
        plsc.subcore_barrier()
        pltpu.sync_copy(deg_sh.at[pl.ds(r0, RPT)], out_hbm.at[c].at[pl.ds(r0, RPT)])

    return k


@functools.lru_cache(maxsize=None)
def _agg_kernel(ws):
    @functools.partial(
        pl.kernel, mesh=_mesh(), compiler_params=_sc_params(),
        out_type=jax.ShapeDtypeStruct((NC, PAD_N, W), jnp.float32),
        scratch_types=[
            pltpu.VMEM((CPW, 2, CHUNK), jnp.int32),
            pltpu.VMEM((CHUNK, ws), jnp.float32),
            pltpu.VMEM((CHUNK, ws), jnp.float32),
            pltpu.VMEM_SHARED((PAD_N, ws), jnp.float32),
            pltpu.VMEM_SHARED((PAD_N, ws), jnp.float32),
            pltpu.SemaphoreType.DMA,
            pltpu.SemaphoreType.DMA,
        ],
    )
    def k(hp_hbm, eidx_hbm, zeros_hbm, out_hbm,
          ev, rows0, rows1, hp_sh, agg_sh, sem0, sem1):
        c = lax.axis_index("c")
        s = lax.axis_index("s")
        wid = c * NS + s
        r0 = s * RPT
        rows = (rows0, rows1)
        sem = (sem0, sem1)
        pltpu.sync_copy(zeros_hbm.at[pl.ds(r0, RPT)], agg_sh.at[pl.ds(r0, RPT)])
        pltpu.sync_copy(hp_hbm.at[pl.ds(r0, RPT), pl.ds(0, ws)],
                        hp_sh.at[pl.ds(r0, RPT)])
        pltpu.sync_copy(eidx_hbm.at[pl.ds(wid * CPW, CPW)], ev)
        plsc.subcore_barrier()

        pltpu.async_copy(hp_sh.at[ev.at[0].at[0]], rows0, sem0)

        def body(i, carry):
            j = i * 2
            for b in range(2):
                cid = j + b
                nxt = cid + 1

                @pl.when(nxt < CPW)
                def _():
                    pltpu.async_copy(hp_sh.at[ev.at[nxt].at[0]],
                                     rows[1 - b], sem[1 - b])

                pltpu.make_async_copy(hp_sh.at[ev.at[cid].at[0]],
                                      rows[b], sem[b]).wait()
                pltpu.sync_copy(rows[b], agg_sh.at[ev.at[cid].at[1]], add=True)
            return carry

        lax.fori_loop(0, CPW // 2, body, 0)
        plsc.subcore_barrier()
        pltpu.sync_copy(agg_sh.at[pl.ds(r0, RPT)],
                        out_hbm.at[c].at[pl.ds(r0, RPT), pl.ds(0, ws)])

    return k


def _tc_scale_matmul(x_p, W1, degt):
    BN = 2048

    def body(x_ref, w_ref, deg_ref, hp_ref, dinv_ref):
        deg = deg_ref[:, 0:1] + deg_ref[:, 1:2] + 1.0
        dinv = lax.rsqrt(deg)
        h = jnp.dot(x_ref[...], w_ref[...], preferred_element_type=jnp.float32)
        hp_ref[...] = jnp.pad(h * dinv, ((0, 0), (0, W - H)))
        dinv_ref[...] = dinv

    return pl.pallas_call(
        body,
        grid=(PAD_N // BN,),
        in_specs=[
            pl.BlockSpec((BN, DF), lambda i: (i, 0)),
            pl.BlockSpec((DF, H), lambda i: (0, 0)),
            pl.BlockSpec((BN, NC), lambda i: (i, 0)),
        ],
        out_specs=[
            pl.BlockSpec((BN, W), lambda i: (i, 0)),
            pl.BlockSpec((BN, 1), lambda i: (i, 0)),
        ],
        out_shape=[
            jax.ShapeDtypeStruct((PAD_N, W), jnp.float32),
            jax.ShapeDtypeStruct((PAD_N, 1), jnp.float32),
        ],
    )(x_p, W1, degt)


def _tc_mid(aggp, hp, dinv, b1, W2):
    BN = 2048

    def body(a_ref, hp_ref, dinv_ref, b1_ref, w2_ref, out_ref):
        p = (a_ref[0] + a_ref[1] + hp_ref[...])[:, :H]
        h1 = jnp.maximum(dinv_ref[...] * p + b1_ref[...], 0.0)
        h2 = jnp.dot(h1, w2_ref[...], preferred_element_type=jnp.float32)
        out_ref[...] = jnp.pad(h2 * dinv_ref[...], ((0, 0), (0, W - C)))

    return pl.pallas_call(
        body,
        grid=(PAD_N // BN,),
        in_specs=[
            pl.BlockSpec((NC, BN, W), lambda i: (0, i, 0)),
            pl.BlockSpec((BN, W), lambda i: (i, 0)),
            pl.BlockSpec((BN, 1), lambda i: (i, 0)),
            pl.BlockSpec((1, H), lambda i: (0, 0)),
            pl.BlockSpec((H, C), lambda i: (0, 0)),
        ],
        out_specs=pl.BlockSpec((BN, W), lambda i: (i, 0)),
        out_shape=jax.ShapeDtypeStruct((PAD_N, W), jnp.float32),
    )(aggp, hp, dinv, b1, W2)


def _tc_final(aggp, h2p, dinv, b2):
    BN = 2048

    def body(a_ref, hp_ref, dinv_ref, b2_ref, out_ref):
        q = (a_ref[0] + a_ref[1] + hp_ref[...])[:, :C]
        z = dinv_ref[...] * q + b2_ref[...]
        m = jnp.max(z, axis=1, keepdims=True)
        e = jnp.exp(z - m)
        out_ref[...] = (z - m) - jnp.log(jnp.sum(e, axis=1, keepdims=True))

    return pl.pallas_call(
        body,
        grid=(PAD_N // BN,),
        in_specs=[
            pl.BlockSpec((NC, BN, W), lambda i: (0, i, 0)),
            pl.BlockSpec((BN, W), lambda i: (i, 0)),
            pl.BlockSpec((BN, 1), lambda i: (i, 0)),
            pl.BlockSpec((1, C), lambda i: (0, 0)),
        ],
        out_specs=pl.BlockSpec((BN, C), lambda i: (i, 0)),
        out_shape=jax.ShapeDtypeStruct((PAD_N, C), jnp.float32),
    )(aggp, h2p, dinv, b2)


def kernel(x, edge_index, W1, b1, W2, b2):
    ei = edge_index.astype(jnp.int32)
    pad_e = E_PAD - E
    srcc = jnp.concatenate(
        [ei[0], jnp.zeros((pad_e,), jnp.int32)]).reshape(NCH, 1, CHUNK)
    dstc = jnp.concatenate(
        [ei[1], jnp.full((pad_e,), TRASH, jnp.int32)]).reshape(NCH, 1, CHUNK)
    eidx = jnp.concatenate([srcc, dstc], axis=1)
    x_p = jnp.pad(x, ((0, PAD_N - N), (0, 0)))
    z1 = jnp.zeros((PAD_N,), jnp.float32)
    za = jnp.zeros((PAD_N, 24), jnp.float32)
    zb = jnp.zeros((PAD_N, 16), jnp.float32)

    degp = _deg_kernel()(dstc.reshape(NCH, CHUNK), z1)
    degt = degp.T
    hp, dinv = _tc_scale_matmul(x_p, W1, degt)
    agg1 = _agg_kernel(24)(hp, eidx, za)
    h2p = _tc_mid(agg1, hp, dinv, b1.reshape(1, H), W2)
    agg2 = _agg_kernel(16)(h2p, eidx, zb)
    out = _tc_final(agg2, h2p, dinv, b2.reshape(1, C))
    return out[:N]


# speedup vs baseline: 7.6086x; 1.0944x over previous
import functools

import jax
import jax.numpy as jnp
from jax import lax
from jax.experimental import pallas as pl
from jax.experimental.pallas import tpu as pltpu
from jax.experimental.pallas import tpu_sc as plsc

N = 10000
E = 320000
DF = 128
H = 20
C = 16
W = 128

NC, NS = 2, 16
NW = NC * NS
PAD_N = 10240
CHUNK = 128
NCHR = E // CHUNK
CPB = NCHR // NW
XTR = NCHR - NW * CPB
RPT = PAD_N // NS

_mesh = lambda: plsc.VectorSubcoreMesh(core_axis_name="c", subcore_axis_name="s")
_sc_params = lambda: pltpu.CompilerParams(use_tc_tiling_on_sc=False, skip_device_barrier=True)


@functools.lru_cache(maxsize=None)
def _deg_kernel():
    @functools.partial(
        pl.kernel, mesh=_mesh(), compiler_params=_sc_params(),
        out_type=jax.ShapeDtypeStruct((NC, PAD_N), jnp.float32),
        scratch_types=[
            pltpu.VMEM((CPB, CHUNK), jnp.int32),
            pltpu.VMEM((1, CHUNK), jnp.int32),
            pltpu.VMEM((CHUNK,), jnp.float32),
            pltpu.VMEM_SHARED((PAD_N,), jnp.float32),
            pltpu.SemaphoreType.DMA,
        ],
    )
    def k(ei_hbm, zeros_hbm, out_hbm, dst_v, dst_x, ones_v, deg_sh, sem):
        c = lax.axis_index("c")
        s = lax.axis_index("s")
        wid = c * NS + s
        for i in range(CHUNK // 16):
            ones_v[pl.ds(i * 16, 16)] = jnp.ones((16,), jnp.float32)
        r0 = s * RPT
        pltpu.sync_copy(zeros_hbm.at[pl.ds(r0, RPT)], deg_sh.at[pl.ds(r0, RPT)])
        pltpu.sync_copy(ei_hbm.at[1].at[pl.ds(wid * CPB, CPB)], dst_v)

        @pl.when(wid < XTR)
        def _():
            pltpu.sync_copy(ei_hbm.at[1].at[pl.ds(NW * CPB + wid, 1)], dst_x)

        plsc.subcore_barrier()

        def body(j, carry):
            pltpu.sync_copy(ones_v, deg_sh.at[dst_v.at[j]], add=True)
            return carry

        lax.fori_loop(0, CPB, body, 0)

        @pl.when(wid < XTR)
        def _():
            pltpu.sync_copy(ones_v, deg_sh.at[dst_x.at[0]], add=True)

        plsc.subcore_barrier()
        pltpu.sync_copy(deg_sh.at[pl.ds(r0, RPT)], out_hbm.at[c].at[pl.ds(r0, RPT)])

    return k


@functools.lru_cache(maxsize=None)
def _agg_kernel(ws):
    @functools.partial(
        pl.kernel, mesh=_mesh(), compiler_params=_sc_params(),
        out_type=jax.ShapeDtypeStruct((NC, PAD_N, W), jnp.float32),
        scratch_types=[
            pltpu.VMEM((CPB, CHUNK), jnp.int32),
            pltpu.VMEM((CPB, CHUNK), jnp.int32),
            pltpu.VMEM((1, CHUNK), jnp.int32),
            pltpu.VMEM((1, CHUNK), jnp.int32),
            pltpu.VMEM((CHUNK, ws), jnp.float32),
            pltpu.VMEM((CHUNK, ws), jnp.float32),
            pltpu.VMEM_SHARED((PAD_N, ws), jnp.float32),
            pltpu.VMEM_SHARED((PAD_N, ws), jnp.float32),
            pltpu.SemaphoreType.DMA,
            pltpu.SemaphoreType.DMA,
        ],
    )
    def k(hp_hbm, ei_hbm, zeros_hbm, out_hbm,
          src_v, dst_v, src_x, dst_x, rows0, rows1, hp_sh, agg_sh, sem0, sem1):
        c = lax.axis_index("c")
        s = lax.axis_index("s")
        wid = c * NS + s
        r0 = s * RPT
        rows = (rows0, rows1)
        sem = (sem0, sem1)
        pltpu.sync_copy(zeros_hbm.at[pl.ds(r0, RPT)], agg_sh.at[pl.ds(r0, RPT)])
        pltpu.sync_copy(hp_hbm.at[pl.ds(r0, RPT), pl.ds(0, ws)],
                        hp_sh.at[pl.ds(r0, RPT)])
        pltpu.sync_copy(ei_hbm.at[0].at[pl.ds(wid * CPB, CPB)], src_v)
        pltpu.sync_copy(ei_hbm.at[1].at[pl.ds(wid * CPB, CPB)], dst_v)

        @pl.when(wid < XTR)
        def _():
            pltpu.sync_copy(ei_hbm.at[0].at[pl.ds(NW * CPB + wid, 1)], src_x)
            pltpu.sync_copy(ei_hbm.at[1].at[pl.ds(NW * CPB + wid, 1)], dst_x)

        plsc.subcore_barrier()

        pltpu.async_copy(hp_sh.at[src_v.at[0]], rows0, sem0)

        def body(i, carry):
            j = i * 2
            for b in range(2):
                cid = j + b
                nxt = cid + 1

                @pl.when(nxt < CPB)
                def _():
                    pltpu.async_copy(hp_sh.at[src_v.at[nxt]],
                                     rows[1 - b], sem[1 - b])

                pltpu.make_async_copy(hp_sh.at[src_v.at[cid]],
                                      rows[b], sem[b]).wait()
                pltpu.sync_copy(rows[b], agg_sh.at[dst_v.at[cid]], add=True)
            return carry

        lax.fori_loop(0, CPB // 2, body, 0)

        @pl.when(wid < XTR)
        def _():
            pltpu.async_copy(hp_sh.at[src_x.at[0]], rows0, sem0).wait()
            pltpu.sync_copy(rows0, agg_sh.at[dst_x.at[0]], add=True)

        plsc.subcore_barrier()
        pltpu.sync_copy(agg_sh.at[pl.ds(r0, RPT)],
                        out_hbm.at[c].at[pl.ds(r0, RPT), pl.ds(0, ws)])

    return k


def _tc_scale_matmul(x_p, W1, degt):
    BN = 2048

    def body(x_ref, w_ref, deg_ref, hp_ref, dinv_ref):
        deg = deg_ref[:, 0:1] + deg_ref[:, 1:2] + 1.0
        dinv = lax.rsqrt(deg)
        h = jnp.dot(x_ref[...], w_ref[...], preferred_element_type=jnp.float32)
        hp_ref[...] = jnp.pad(h * dinv, ((0, 0), (0, W - H)))
        dinv_ref[...] = dinv

    return pl.pallas_call(
        body,
        grid=(PAD_N // BN,),
        in_specs=[
            pl.BlockSpec((BN, DF), lambda i: (i, 0)),
            pl.BlockSpec((DF, H), lambda i: (0, 0)),
            pl.BlockSpec((BN, NC), lambda i: (i, 0)),
        ],
        out_specs=[
            pl.BlockSpec((BN, W), lambda i: (i, 0)),
            pl.BlockSpec((BN, 1), lambda i: (i, 0)),
        ],
        out_shape=[
            jax.ShapeDtypeStruct((PAD_N, W), jnp.float32),
            jax.ShapeDtypeStruct((PAD_N, 1), jnp.float32),
        ],
    )(x_p, W1, degt)


def _tc_mid(aggp, hp, dinv, b1, W2):
    BN = 2048

    def body(a_ref, hp_ref, dinv_ref, b1_ref, w2_ref, out_ref):
        p = (a_ref[0] + a_ref[1] + hp_ref[...])[:, :H]
        h1 = jnp.maximum(dinv_ref[...] * p + b1_ref[...], 0.0)
        h2 = jnp.dot(h1, w2_ref[...], preferred_element_type=jnp.float32)
        out_ref[...] = jnp.pad(h2 * dinv_ref[...], ((0, 0), (0, W - C)))

    return pl.pallas_call(
        body,
        grid=(PAD_N // BN,),
        in_specs=[
            pl.BlockSpec((NC, BN, W), lambda i: (0, i, 0)),
            pl.BlockSpec((BN, W), lambda i: (i, 0)),
            pl.BlockSpec((BN, 1), lambda i: (i, 0)),
            pl.BlockSpec((1, H), lambda i: (0, 0)),
            pl.BlockSpec((H, C), lambda i: (0, 0)),
        ],
        out_specs=pl.BlockSpec((BN, W), lambda i: (i, 0)),
        out_shape=jax.ShapeDtypeStruct((PAD_N, W), jnp.float32),
    )(aggp, hp, dinv, b1, W2)


def _tc_final(aggp, h2p, dinv, b2):
    BN = 2048

    def body(a_ref, hp_ref, dinv_ref, b2_ref, out_ref):
        q = (a_ref[0] + a_ref[1] + hp_ref[...])[:, :C]
        z = dinv_ref[...] * q + b2_ref[...]
        m = jnp.max(z, axis=1, keepdims=True)
        e = jnp.exp(z - m)
        out_ref[...] = (z - m) - jnp.log(jnp.sum(e, axis=1, keepdims=True))

    return pl.pallas_call(
        body,
        grid=(PAD_N // BN,),
        in_specs=[
            pl.BlockSpec((NC, BN, W), lambda i: (0, i, 0)),
            pl.BlockSpec((BN, W), lambda i: (i, 0)),
            pl.BlockSpec((BN, 1), lambda i: (i, 0)),
            pl.BlockSpec((1, C), lambda i: (0, 0)),
        ],
        out_specs=pl.BlockSpec((BN, C), lambda i: (i, 0)),
        out_shape=jax.ShapeDtypeStruct((PAD_N, C), jnp.float32),
    )(aggp, h2p, dinv, b2)


def kernel(x, edge_index, W1, b1, W2, b2):
    ei3 = edge_index.astype(jnp.int32).reshape(2, NCHR, CHUNK)
    x_p = jnp.pad(x, ((0, PAD_N - N), (0, 0)))
    z1 = jnp.zeros((PAD_N,), jnp.float32)
    za = jnp.zeros((PAD_N, 24), jnp.float32)
    zb = jnp.zeros((PAD_N, 16), jnp.float32)

    degp = _deg_kernel()(ei3, z1)
    degt = degp.T
    hp, dinv = _tc_scale_matmul(x_p, W1, degt)
    agg1 = _agg_kernel(24)(hp, ei3, za)
    h2p = _tc_mid(agg1, hp, dinv, b1.reshape(1, H), W2)
    agg2 = _agg_kernel(16)(h2p, ei3, zb)
    out = _tc_final(agg2, h2p, dinv, b2.reshape(1, C))
    return out[:N]


# speedup vs baseline: 7.7216x; 1.0148x over previous
import functools

import jax
import jax.numpy as jnp
from jax import lax
from jax.experimental import pallas as pl
from jax.experimental.pallas import tpu as pltpu
from jax.experimental.pallas import tpu_sc as plsc

N = 10000
E = 320000
DF = 128
H = 20
C = 16
W = 128

NC, NS = 2, 16
NW = NC * NS
PAD_N = 10240
CHUNK = 128
NCHR = E // CHUNK
CPB = NCHR // NW
XTR = NCHR - NW * CPB
RPT = PAD_N // NS

_mesh = lambda: plsc.VectorSubcoreMesh(core_axis_name="c", subcore_axis_name="s")
_sc_params = lambda: pltpu.CompilerParams(use_tc_tiling_on_sc=False, skip_device_barrier=True)


@functools.lru_cache(maxsize=None)
def _deg_kernel():
    @functools.partial(
        pl.kernel, mesh=_mesh(), compiler_params=_sc_params(),
        out_type=jax.ShapeDtypeStruct((NC, PAD_N), jnp.float32),
        scratch_types=[
            pltpu.VMEM((CPB, CHUNK), jnp.int32),
            pltpu.VMEM((1, CHUNK), jnp.int32),
            pltpu.VMEM((CHUNK,), jnp.float32),
            pltpu.VMEM_SHARED((PAD_N,), jnp.float32),
            pltpu.SemaphoreType.DMA,
        ],
    )
    def k(ei_hbm, zeros_hbm, out_hbm, dst_v, dst_x, ones_v, deg_sh, sem):
        c = lax.axis_index("c")
        s = lax.axis_index("s")
        wid = c * NS + s
        for i in range(CHUNK // 16):
            ones_v[pl.ds(i * 16, 16)] = jnp.ones((16,), jnp.float32)
        r0 = s * RPT
        pltpu.sync_copy(zeros_hbm.at[pl.ds(r0, RPT)], deg_sh.at[pl.ds(r0, RPT)])
        pltpu.sync_copy(ei_hbm.at[1].at[pl.ds(wid * CPB, CPB)], dst_v)

        @pl.when(wid < XTR)
        def _():
            pltpu.sync_copy(ei_hbm.at[1].at[pl.ds(NW * CPB + wid, 1)], dst_x)

        plsc.subcore_barrier()

        def body(j, carry):
            pltpu.sync_copy(ones_v, deg_sh.at[dst_v.at[j]], add=True)
            return carry

        lax.fori_loop(0, CPB, body, 0)

        @pl.when(wid < XTR)
        def _():
            pltpu.sync_copy(ones_v, deg_sh.at[dst_x.at[0]], add=True)

        plsc.subcore_barrier()
        pltpu.sync_copy(deg_sh.at[pl.ds(r0, RPT)], out_hbm.at[c].at[pl.ds(r0, RPT)])

    return k


@functools.lru_cache(maxsize=None)
def _agg_kernel(ws):
    @functools.partial(
        pl.kernel, mesh=_mesh(), compiler_params=_sc_params(),
        out_type=jax.ShapeDtypeStruct((NC, PAD_N, W), jnp.float32),
        scratch_types=[
            pltpu.VMEM((CPB, CHUNK), jnp.int32),
            pltpu.VMEM((CPB, CHUNK), jnp.int32),
            pltpu.VMEM((1, CHUNK), jnp.int32),
            pltpu.VMEM((1, CHUNK), jnp.int32),
            pltpu.VMEM((CHUNK, ws), jnp.float32),
            pltpu.VMEM((CHUNK, ws), jnp.float32),
            pltpu.VMEM_SHARED((PAD_N, ws), jnp.float32),
            pltpu.VMEM_SHARED((PAD_N, ws), jnp.float32),
            pltpu.SemaphoreType.DMA,
            pltpu.SemaphoreType.DMA,
        ],
    )
    def k(hp_hbm, ei_hbm, zeros_hbm, out_hbm,
          src_v, dst_v, src_x, dst_x, rows0, rows1, hp_sh, agg_sh, sem0, sem1):
        c = lax.axis_index("c")
        s = lax.axis_index("s")
        wid = c * NS + s
        r0 = s * RPT
        rows = (rows0, rows1)
        sem = (sem0, sem1)
        pltpu.sync_copy(zeros_hbm.at[pl.ds(r0, RPT)], agg_sh.at[pl.ds(r0, RPT)])
        pltpu.sync_copy(hp_hbm.at[pl.ds(r0, RPT), pl.ds(0, ws)],
                        hp_sh.at[pl.ds(r0, RPT)])
        pltpu.sync_copy(ei_hbm.at[0].at[pl.ds(wid * CPB, CPB)], src_v)
        pltpu.sync_copy(ei_hbm.at[1].at[pl.ds(wid * CPB, CPB)], dst_v)

        @pl.when(wid < XTR)
        def _():
            pltpu.sync_copy(ei_hbm.at[0].at[pl.ds(NW * CPB + wid, 1)], src_x)
            pltpu.sync_copy(ei_hbm.at[1].at[pl.ds(NW * CPB + wid, 1)], dst_x)

        plsc.subcore_barrier()

        pltpu.async_copy(hp_sh.at[src_v.at[0]], rows0, sem0)

        def body(i, carry):
            j = i * 2
            for b in range(2):
                cid = j + b
                nxt = cid + 1

                @pl.when(nxt < CPB)
                def _():
                    pltpu.async_copy(hp_sh.at[src_v.at[nxt]],
                                     rows[1 - b], sem[1 - b])

                pltpu.make_async_copy(hp_sh.at[src_v.at[cid]],
                                      rows[b], sem[b]).wait()
                pltpu.sync_copy(rows[b], agg_sh.at[dst_v.at[cid]], add=True)
            return carry

        lax.fori_loop(0, CPB // 2, body, 0)

        @pl.when(wid < XTR)
        def _():
            pltpu.async_copy(hp_sh.at[src_x.at[0]], rows0, sem0).wait()
            pltpu.sync_copy(rows0, agg_sh.at[dst_x.at[0]], add=True)

        plsc.subcore_barrier()
        pltpu.sync_copy(agg_sh.at[pl.ds(r0, RPT)],
                        out_hbm.at[c].at[pl.ds(r0, RPT), pl.ds(0, ws)])

    return k


def _tc_scale_matmul(x_p, W1, degt):
    BN = 2048

    def body(x_ref, w_ref, deg_ref, hp_ref, dinv_ref):
        deg = deg_ref[:, 0:1] + deg_ref[:, 1:2] + 1.0
        dinv = lax.rsqrt(deg)
        h = jnp.dot(x_ref[...], w_ref[...], preferred_element_type=jnp.float32)
        hp_ref[...] = jnp.pad(h * dinv, ((0, 0), (0, W - H)))
        dinv_ref[...] = dinv

    return pl.pallas_call(
        body,
        grid=(PAD_N // BN,),
        in_specs=[
            pl.BlockSpec((BN, DF), lambda i: (i, 0)),
            pl.BlockSpec((DF, H), lambda i: (0, 0)),
            pl.BlockSpec((BN, NC), lambda i: (i, 0)),
        ],
        out_specs=[
            pl.BlockSpec((BN, W), lambda i: (i, 0)),
            pl.BlockSpec((BN, 1), lambda i: (i, 0)),
        ],
        out_shape=[
            jax.ShapeDtypeStruct((PAD_N, W), jnp.float32),
            jax.ShapeDtypeStruct((PAD_N, 1), jnp.float32),
        ],
    )(x_p, W1, degt)


def _tc_mid(aggp, hp, dinv, b1, W2):
    BN = 2048

    def body(a_ref, hp_ref, dinv_ref, b1_ref, w2_ref, out_ref):
        p = (a_ref[0] + a_ref[1] + hp_ref[...])[:, :H]
        h1 = jnp.maximum(dinv_ref[...] * p + b1_ref[...], 0.0)
        h2 = jnp.dot(h1, w2_ref[...], preferred_element_type=jnp.float32)
        out_ref[...] = jnp.pad(h2 * dinv_ref[...], ((0, 0), (0, W - C)))

    return pl.pallas_call(
        body,
        grid=(PAD_N // BN,),
        in_specs=[
            pl.BlockSpec((NC, BN, W), lambda i: (0, i, 0)),
            pl.BlockSpec((BN, W), lambda i: (i, 0)),
            pl.BlockSpec((BN, 1), lambda i: (i, 0)),
            pl.BlockSpec((1, H), lambda i: (0, 0)),
            pl.BlockSpec((H, C), lambda i: (0, 0)),
        ],
        out_specs=pl.BlockSpec((BN, W), lambda i: (i, 0)),
        out_shape=jax.ShapeDtypeStruct((PAD_N, W), jnp.float32),
    )(aggp, hp, dinv, b1, W2)


def _tc_final(aggp, h2p, dinv, b2):
    BN = 2048

    def body(a_ref, hp_ref, dinv_ref, b2_ref, out_ref):
        q = (a_ref[0] + a_ref[1] + hp_ref[...])[:, :C]
        z = dinv_ref[...] * q + b2_ref[...]
        m = jnp.max(z, axis=1, keepdims=True)
        e = jnp.exp(z - m)
        out_ref[...] = (z - m) - jnp.log(jnp.sum(e, axis=1, keepdims=True))

    return pl.pallas_call(
        body,
        grid=(PAD_N // BN,),
        in_specs=[
            pl.BlockSpec((NC, BN, W), lambda i: (0, i, 0)),
            pl.BlockSpec((BN, W), lambda i: (i, 0)),
            pl.BlockSpec((BN, 1), lambda i: (i, 0)),
            pl.BlockSpec((1, C), lambda i: (0, 0)),
        ],
        out_specs=pl.BlockSpec((BN, C), lambda i: (i, 0)),
        out_shape=jax.ShapeDtypeStruct((PAD_N, C), jnp.float32),
    )(aggp, h2p, dinv, b2)


def kernel(x, edge_index, W1, b1, W2, b2):
    ei3 = edge_index.astype(jnp.int32).reshape(2, NCHR, CHUNK)
    z1 = jnp.zeros((PAD_N,), jnp.float32)
    za = jnp.zeros((PAD_N, 24), jnp.float32)
    zb = jnp.zeros((PAD_N, 16), jnp.float32)

    degp = _deg_kernel()(ei3, z1)
    degt = degp.T
    hp, dinv = _tc_scale_matmul(x, W1, degt)
    agg1 = _agg_kernel(24)(hp, ei3, za)
    h2p = _tc_mid(agg1, hp, dinv, b1.reshape(1, H), W2)
    agg2 = _agg_kernel(16)(h2p, ei3, zb)
    out = _tc_final(agg2, h2p, dinv, b2.reshape(1, C))
    return out[:N]


# speedup vs baseline: 7.7825x; 1.0079x over previous
import functools

import jax
import jax.numpy as jnp
from jax import lax
from jax.experimental import pallas as pl
from jax.experimental.pallas import tpu as pltpu
from jax.experimental.pallas import tpu_sc as plsc

N = 10000
E = 320000
DF = 128
H = 20
C = 16
W = 128

NC, NS = 2, 16
NW = NC * NS
PAD_N = 10240
CHUNK = 128
NCHR = E // CHUNK
CPB = NCHR // NW
XTR = NCHR - NW * CPB
RPT = PAD_N // NS

_mesh = lambda: plsc.VectorSubcoreMesh(core_axis_name="c", subcore_axis_name="s")
_sc_params = lambda: pltpu.CompilerParams(use_tc_tiling_on_sc=False, skip_device_barrier=True)


@functools.lru_cache(maxsize=None)
def _deg_kernel():
    @functools.partial(
        pl.kernel, mesh=_mesh(), compiler_params=_sc_params(),
        out_type=jax.ShapeDtypeStruct((NC, PAD_N), jnp.float32),
        scratch_types=[
            pltpu.VMEM((CPB, CHUNK), jnp.int32),
            pltpu.VMEM((1, CHUNK), jnp.int32),
            pltpu.VMEM((CHUNK,), jnp.float32),
            pltpu.VMEM_SHARED((PAD_N,), jnp.float32),
            pltpu.SemaphoreType.DMA,
        ],
    )
    def k(ei_hbm, zeros_hbm, out_hbm, dst_v, dst_x, ones_v, deg_sh, sem):
        c = lax.axis_index("c")
        s = lax.axis_index("s")
        wid = c * NS + s
        for i in range(CHUNK // 16):
            ones_v[pl.ds(i * 16, 16)] = jnp.ones((16,), jnp.float32)
        r0 = s * RPT
        pltpu.sync_copy(zeros_hbm.at[pl.ds(r0, RPT)], deg_sh.at[pl.ds(r0, RPT)])
        pltpu.sync_copy(ei_hbm.at[1].at[pl.ds(wid * CPB, CPB)], dst_v)

        @pl.when(wid < XTR)
        def _():
            pltpu.sync_copy(ei_hbm.at[1].at[pl.ds(NW * CPB + wid, 1)], dst_x)

        plsc.subcore_barrier()

        def body(j, carry):
            pltpu.sync_copy(ones_v, deg_sh.at[dst_v.at[j]], add=True)
            return carry

        lax.fori_loop(0, CPB, body, 0)

        @pl.when(wid < XTR)
        def _():
            pltpu.sync_copy(ones_v, deg_sh.at[dst_x.at[0]], add=True)

        plsc.subcore_barrier()
        pltpu.sync_copy(deg_sh.at[pl.ds(r0, RPT)], out_hbm.at[c].at[pl.ds(r0, RPT)])

    return k


@functools.lru_cache(maxsize=None)
def _agg_kernel(ws):
    @functools.partial(
        pl.kernel, mesh=_mesh(), compiler_params=_sc_params(),
        out_type=jax.ShapeDtypeStruct((NC, PAD_N, W), jnp.float32),
        scratch_types=[
            pltpu.VMEM((CPB, CHUNK), jnp.int32),
            pltpu.VMEM((CPB, CHUNK), jnp.int32),
            pltpu.VMEM((1, CHUNK), jnp.int32),
            pltpu.VMEM((1, CHUNK), jnp.int32),
            pltpu.VMEM((CHUNK, ws), jnp.float32),
            pltpu.VMEM((CHUNK, ws), jnp.float32),
            pltpu.VMEM_SHARED((PAD_N, ws), jnp.float32),
            pltpu.VMEM_SHARED((PAD_N, ws), jnp.float32),
            pltpu.SemaphoreType.DMA,
            pltpu.SemaphoreType.DMA,
        ],
    )
    def k(hp_hbm, ei_hbm, zeros_hbm, out_hbm,
          src_v, dst_v, src_x, dst_x, rows0, rows1, hp_sh, agg_sh, sem0, sem1):
        c = lax.axis_index("c")
        s = lax.axis_index("s")
        wid = c * NS + s
        r0 = s * RPT
        rows = (rows0, rows1)
        sem = (sem0, sem1)
        pltpu.sync_copy(zeros_hbm.at[pl.ds(r0, RPT)], agg_sh.at[pl.ds(r0, RPT)])
        pltpu.sync_copy(hp_hbm.at[pl.ds(r0, RPT), pl.ds(0, ws)],
                        hp_sh.at[pl.ds(r0, RPT)])
        pltpu.sync_copy(ei_hbm.at[0].at[pl.ds(wid * CPB, CPB)], src_v)
        pltpu.sync_copy(ei_hbm.at[1].at[pl.ds(wid * CPB, CPB)], dst_v)

        @pl.when(wid < XTR)
        def _():
            pltpu.sync_copy(ei_hbm.at[0].at[pl.ds(NW * CPB + wid, 1)], src_x)
            pltpu.sync_copy(ei_hbm.at[1].at[pl.ds(NW * CPB + wid, 1)], dst_x)

        plsc.subcore_barrier()

        pltpu.async_copy(hp_sh.at[src_v.at[0]], rows0, sem0)

        def body(i, carry):
            j = i * 2
            for b in range(2):
                cid = j + b
                nxt = cid + 1

                @pl.when(nxt < CPB)
                def _():
                    pltpu.async_copy(hp_sh.at[src_v.at[nxt]],
                                     rows[1 - b], sem[1 - b])

                pltpu.make_async_copy(hp_sh.at[src_v.at[cid]],
                                      rows[b], sem[b]).wait()
                pltpu.sync_copy(rows[b], agg_sh.at[dst_v.at[cid]], add=True)
            return carry

        lax.fori_loop(0, CPB // 2, body, 0)

        @pl.when(wid < XTR)
        def _():
            pltpu.async_copy(hp_sh.at[src_x.at[0]], rows0, sem0).wait()
            pltpu.sync_copy(rows0, agg_sh.at[dst_x.at[0]], add=True)

        plsc.subcore_barrier()
        pltpu.sync_copy(agg_sh.at[pl.ds(r0, RPT)],
                        out_hbm.at[c].at[pl.ds(r0, RPT), pl.ds(0, ws)])

    return k


def _tc_scale_matmul(x_p, W1, degt):
    BN = 2048

    def body(x_ref, w_ref, deg_ref, hp_ref, dinv_ref):
        deg = deg_ref[:, 0:1] + deg_ref[:, 1:2] + 1.0
        dinv = lax.rsqrt(deg)
        h = jnp.dot(x_ref[...], w_ref[...], preferred_element_type=jnp.float32)
        hp_ref[...] = jnp.pad(h * dinv, ((0, 0), (0, W - H)))
        dinv_ref[...] = dinv

    return pl.pallas_call(
        body,
        grid=(PAD_N // BN,),
        in_specs=[
            pl.BlockSpec((BN, DF), lambda i: (i, 0)),
            pl.BlockSpec((DF, H), lambda i: (0, 0)),
            pl.BlockSpec((BN, NC), lambda i: (i, 0)),
        ],
        out_specs=[
            pl.BlockSpec((BN, W), lambda i: (i, 0)),
            pl.BlockSpec((BN, 1), lambda i: (i, 0)),
        ],
        out_shape=[
            jax.ShapeDtypeStruct((PAD_N, W), jnp.float32),
            jax.ShapeDtypeStruct((PAD_N, 1), jnp.float32),
        ],
    )(x_p, W1, degt)


def _tc_mid(aggp, hp, dinv, b1, W2):
    BN = 2048

    def body(a_ref, hp_ref, dinv_ref, b1_ref, w2_ref, out_ref):
        p = (a_ref[0] + a_ref[1] + hp_ref[...])[:, :H]
        h1 = jnp.maximum(dinv_ref[...] * p + b1_ref[...], 0.0)
        h2 = jnp.dot(h1, w2_ref[...], preferred_element_type=jnp.float32)
        out_ref[...] = jnp.pad(h2 * dinv_ref[...], ((0, 0), (0, W - C)))

    return pl.pallas_call(
        body,
        grid=(PAD_N // BN,),
        in_specs=[
            pl.BlockSpec((NC, BN, W), lambda i: (0, i, 0)),
            pl.BlockSpec((BN, W), lambda i: (i, 0)),
            pl.BlockSpec((BN, 1), lambda i: (i, 0)),
            pl.BlockSpec((1, H), lambda i: (0, 0)),
            pl.BlockSpec((H, C), lambda i: (0, 0)),
        ],
        out_specs=pl.BlockSpec((BN, W), lambda i: (i, 0)),
        out_shape=jax.ShapeDtypeStruct((PAD_N, W), jnp.float32),
    )(aggp, hp, dinv, b1, W2)


def _tc_final(aggp, h2p, dinv, b2):
    BN = 2048

    def body(a_ref, hp_ref, dinv_ref, b2_ref, out_ref):
        q = (a_ref[0] + a_ref[1] + hp_ref[...])[:, :C]
        z = dinv_ref[...] * q + b2_ref[...]
        m = jnp.max(z, axis=1, keepdims=True)
        e = jnp.exp(z - m)
        out_ref[...] = (z - m) - jnp.log(jnp.sum(e, axis=1, keepdims=True))

    return pl.pallas_call(
        body,
        grid=(PAD_N // BN,),
        in_specs=[
            pl.BlockSpec((NC, BN, W), lambda i: (0, i, 0)),
            pl.BlockSpec((BN, W), lambda i: (i, 0)),
            pl.BlockSpec((BN, 1), lambda i: (i, 0)),
            pl.BlockSpec((1, C), lambda i: (0, 0)),
        ],
        out_specs=pl.BlockSpec((BN, C), lambda i: (i, 0)),
        out_shape=jax.ShapeDtypeStruct((N, C), jnp.float32),
    )(aggp, h2p, dinv, b2)


def kernel(x, edge_index, W1, b1, W2, b2):
    ei3 = edge_index.astype(jnp.int32).reshape(2, NCHR, CHUNK)
    z1 = jnp.zeros((PAD_N,), jnp.float32)
    za = jnp.zeros((PAD_N, 24), jnp.float32)
    zb = jnp.zeros((PAD_N, 16), jnp.float32)

    degp = _deg_kernel()(ei3, z1)
    degt = degp.T
    hp, dinv = _tc_scale_matmul(x, W1, degt)
    agg1 = _agg_kernel(24)(hp, ei3, za)
    h2p = _tc_mid(agg1, hp, dinv, b1.reshape(1, H), W2)
    agg2 = _agg_kernel(16)(h2p, ei3, zb)
    return _tc_final(agg2, h2p, dinv, b2.reshape(1, C))


# speedup vs baseline: 8.2375x; 1.0585x over previous
import functools

import jax
import jax.numpy as jnp
from jax import lax
from jax.experimental import pallas as pl
from jax.experimental.pallas import tpu as pltpu
from jax.experimental.pallas import tpu_sc as plsc

N = 10000
E = 320000
DF = 128
H = 20
C = 16
W = 128

NC, NS = 2, 16
NW = NC * NS
PAD_N = 10240
CHUNK = 128
NCHR = E // CHUNK
CPB = NCHR // NW
XTR = NCHR - NW * CPB
RPT = PAD_N // NS

_mesh = lambda: plsc.VectorSubcoreMesh(core_axis_name="c", subcore_axis_name="s")
_sc_params = lambda: pltpu.CompilerParams(use_tc_tiling_on_sc=False, skip_device_barrier=True)


@functools.lru_cache(maxsize=None)
def _deg_kernel():
    @functools.partial(
        pl.kernel, mesh=_mesh(), compiler_params=_sc_params(),
        out_type=jax.ShapeDtypeStruct((NC, PAD_N), jnp.float32),
        scratch_types=[
            pltpu.VMEM((CPB, CHUNK), jnp.int32),
            pltpu.VMEM((1, CHUNK), jnp.int32),
            pltpu.VMEM((CHUNK,), jnp.float32),
            pltpu.VMEM_SHARED((PAD_N,), jnp.float32),
            pltpu.SemaphoreType.DMA,
        ],
    )
    def k(ei_hbm, zeros_hbm, out_hbm, dst_v, dst_x, ones_v, deg_sh, sem):
        c = lax.axis_index("c")
        s = lax.axis_index("s")
        wid = c * NS + s
        for i in range(CHUNK // 16):
            ones_v[pl.ds(i * 16, 16)] = jnp.ones((16,), jnp.float32)
        r0 = s * RPT
        pltpu.sync_copy(zeros_hbm.at[pl.ds(r0, RPT)], deg_sh.at[pl.ds(r0, RPT)])
        pltpu.sync_copy(ei_hbm.at[1].at[pl.ds(wid * CPB, CPB)], dst_v)

        @pl.when(wid < XTR)
        def _():
            pltpu.sync_copy(ei_hbm.at[1].at[pl.ds(NW * CPB + wid, 1)], dst_x)

        plsc.subcore_barrier()

        def body(j, carry):
            pltpu.sync_copy(ones_v, deg_sh.at[dst_v.at[j]], add=True)
            return carry

        lax.fori_loop(0, CPB, body, 0)

        @pl.when(wid < XTR)
        def _():
            pltpu.sync_copy(ones_v, deg_sh.at[dst_x.at[0]], add=True)

        plsc.subcore_barrier()
        pltpu.sync_copy(deg_sh.at[pl.ds(r0, RPT)], out_hbm.at[c].at[pl.ds(r0, RPT)])

    return k


@functools.lru_cache(maxsize=None)
def _agg_kernel(ws):
    @functools.partial(
        pl.kernel, mesh=_mesh(), compiler_params=_sc_params(),
        out_type=jax.ShapeDtypeStruct((NC, PAD_N, W), jnp.float32),
        scratch_types=[
            pltpu.VMEM((CPB, CHUNK), jnp.int32),
            pltpu.VMEM((CPB, CHUNK), jnp.int32),
            pltpu.VMEM((1, CHUNK), jnp.int32),
            pltpu.VMEM((1, CHUNK), jnp.int32),
            pltpu.VMEM((CHUNK, ws), jnp.float32),
            pltpu.VMEM((CHUNK, ws), jnp.float32),
            pltpu.VMEM((CHUNK, ws), jnp.float32),
            pltpu.VMEM((CHUNK, ws), jnp.float32),
            pltpu.VMEM_SHARED((PAD_N, ws), jnp.float32),
            pltpu.VMEM_SHARED((PAD_N, ws), jnp.float32),
            pltpu.SemaphoreType.DMA,
            pltpu.SemaphoreType.DMA,
            pltpu.SemaphoreType.DMA,
            pltpu.SemaphoreType.DMA,
            pltpu.SemaphoreType.DMA,
            pltpu.SemaphoreType.DMA,
            pltpu.SemaphoreType.DMA,
            pltpu.SemaphoreType.DMA,
        ],
    )
    def k(hp_hbm, ei_hbm, zeros_hbm, out_hbm,
          src_v, dst_v, src_x, dst_x, rows0, rows1, rows2, rows3,
          hp_sh, agg_sh, g0, g1, g2, g3, s0, s1, s2, s3):
        c = lax.axis_index("c")
        s = lax.axis_index("s")
        wid = c * NS + s
        r0 = s * RPT
        rows = (rows0, rows1, rows2, rows3)
        gsem = (g0, g1, g2, g3)
        ssem = (s0, s1, s2, s3)
        pltpu.sync_copy(zeros_hbm.at[pl.ds(r0, RPT)], agg_sh.at[pl.ds(r0, RPT)])
        pltpu.sync_copy(hp_hbm.at[pl.ds(r0, RPT), pl.ds(0, ws)],
                        hp_sh.at[pl.ds(r0, RPT)])
        pltpu.sync_copy(ei_hbm.at[0].at[pl.ds(wid * CPB, CPB)], src_v)
        pltpu.sync_copy(ei_hbm.at[1].at[pl.ds(wid * CPB, CPB)], dst_v)

        @pl.when(wid < XTR)
        def _():
            pltpu.sync_copy(ei_hbm.at[0].at[pl.ds(NW * CPB + wid, 1)], src_x)
            pltpu.sync_copy(ei_hbm.at[1].at[pl.ds(NW * CPB + wid, 1)], dst_x)

        plsc.subcore_barrier()

        pltpu.async_copy(hp_sh.at[src_v.at[0]], rows2, g2)
        pltpu.async_copy(hp_sh.at[src_v.at[1]], rows3, g3)

        def body(i, carry):
            j = i * 4
            for b in range(4):
                cid = j + b

                @pl.when(cid < CPB)
                def _():
                    pre = cid + 2

                    @pl.when(pre < CPB)
                    def _():
                        @pl.when(cid >= 2)
                        def _():
                            pltpu.make_async_copy(
                                rows[b], agg_sh.at[dst_v.at[cid - 2]],
                                ssem[b]).wait()

                        pltpu.async_copy(hp_sh.at[src_v.at[pre]],
                                         rows[b], gsem[b])

                    pltpu.make_async_copy(hp_sh.at[src_v.at[cid]],
                                          rows[(b + 2) % 4],
                                          gsem[(b + 2) % 4]).wait()
                    pltpu.async_copy(rows[(b + 2) % 4],
                                     agg_sh.at[dst_v.at[cid]],
                                     ssem[(b + 2) % 4], add=True)
            return carry

        lax.fori_loop(0, (CPB + 3) // 4, body, 0)
        for t in range(CPB - 4, CPB):
            pltpu.make_async_copy(rows[(t + 2) % 4], agg_sh.at[dst_v.at[t]],
                                  ssem[(t + 2) % 4]).wait()

        @pl.when(wid < XTR)
        def _():
            pltpu.async_copy(hp_sh.at[src_x.at[0]], rows0, g0).wait()
            pltpu.sync_copy(rows0, agg_sh.at[dst_x.at[0]], add=True)

        plsc.subcore_barrier()
        pltpu.sync_copy(agg_sh.at[pl.ds(r0, RPT)],
                        out_hbm.at[c].at[pl.ds(r0, RPT), pl.ds(0, ws)])

    return k


def _tc_scale_matmul(x_p, W1, degt):
    BN = 2048

    def body(x_ref, w_ref, deg_ref, hp_ref, dinv_ref):
        deg = deg_ref[:, 0:1] + deg_ref[:, 1:2] + 1.0
        dinv = lax.rsqrt(deg)
        h = jnp.dot(x_ref[...], w_ref[...], preferred_element_type=jnp.float32)
        hp_ref[...] = jnp.pad(h * dinv, ((0, 0), (0, W - H)))
        dinv_ref[...] = dinv

    return pl.pallas_call(
        body,
        grid=(PAD_N // BN,),
        in_specs=[
            pl.BlockSpec((BN, DF), lambda i: (i, 0)),
            pl.BlockSpec((DF, H), lambda i: (0, 0)),
            pl.BlockSpec((BN, NC), lambda i: (i, 0)),
        ],
        out_specs=[
            pl.BlockSpec((BN, W), lambda i: (i, 0)),
            pl.BlockSpec((BN, 1), lambda i: (i, 0)),
        ],
        out_shape=[
            jax.ShapeDtypeStruct((PAD_N, W), jnp.float32),
            jax.ShapeDtypeStruct((PAD_N, 1), jnp.float32),
        ],
    )(x_p, W1, degt)


def _tc_mid(aggp, hp, dinv, b1, W2):
    BN = 2048

    def body(a_ref, hp_ref, dinv_ref, b1_ref, w2_ref, out_ref):
        p = (a_ref[0] + a_ref[1] + hp_ref[...])[:, :H]
        h1 = jnp.maximum(dinv_ref[...] * p + b1_ref[...], 0.0)
        h2 = jnp.dot(h1, w2_ref[...], preferred_element_type=jnp.float32)
        out_ref[...] = jnp.pad(h2 * dinv_ref[...], ((0, 0), (0, W - C)))

    return pl.pallas_call(
        body,
        grid=(PAD_N // BN,),
        in_specs=[
            pl.BlockSpec((NC, BN, W), lambda i: (0, i, 0)),
            pl.BlockSpec((BN, W), lambda i: (i, 0)),
            pl.BlockSpec((BN, 1), lambda i: (i, 0)),
            pl.BlockSpec((1, H), lambda i: (0, 0)),
            pl.BlockSpec((H, C), lambda i: (0, 0)),
        ],
        out_specs=pl.BlockSpec((BN, W), lambda i: (i, 0)),
        out_shape=jax.ShapeDtypeStruct((PAD_N, W), jnp.float32),
    )(aggp, hp, dinv, b1, W2)


def _tc_final(aggp, h2p, dinv, b2):
    BN = 2048

    def body(a_ref, hp_ref, dinv_ref, b2_ref, out_ref):
        q = (a_ref[0] + a_ref[1] + hp_ref[...])[:, :C]
        z = dinv_ref[...] * q + b2_ref[...]
        m = jnp.max(z, axis=1, keepdims=True)
        e = jnp.exp(z - m)
        out_ref[...] = (z - m) - jnp.log(jnp.sum(e, axis=1, keepdims=True))

    return pl.pallas_call(
        body,
        grid=(PAD_N // BN,),
        in_specs=[
            pl.BlockSpec((NC, BN, W), lambda i: (0, i, 0)),
            pl.BlockSpec((BN, W), lambda i: (i, 0)),
            pl.BlockSpec((BN, 1), lambda i: (i, 0)),
            pl.BlockSpec((1, C), lambda i: (0, 0)),
        ],
        out_specs=pl.BlockSpec((BN, C), lambda i: (i, 0)),
        out_shape=jax.ShapeDtypeStruct((N, C), jnp.float32),
    )(aggp, h2p, dinv, b2)


def kernel(x, edge_index, W1, b1, W2, b2):
    ei3 = edge_index.astype(jnp.int32).reshape(2, NCHR, CHUNK)
    z1 = jnp.zeros((PAD_N,), jnp.float32)
    za = jnp.zeros((PAD_N, 24), jnp.float32)
    zb = jnp.zeros((PAD_N, 16), jnp.float32)

    degp = _deg_kernel()(ei3, z1)
    degt = degp.T
    hp, dinv = _tc_scale_matmul(x, W1, degt)
    agg1 = _agg_kernel(24)(hp, ei3, za)
    h2p = _tc_mid(agg1, hp, dinv, b1.reshape(1, H), W2)
    agg2 = _agg_kernel(16)(h2p, ei3, zb)
    return _tc_final(agg2, h2p, dinv, b2.reshape(1, C))


# speedup vs baseline: 8.5085x; 1.0329x over previous
import functools

import jax
import jax.numpy as jnp
from jax import lax
from jax.experimental import pallas as pl
from jax.experimental.pallas import tpu as pltpu
from jax.experimental.pallas import tpu_sc as plsc

N = 10000
E = 320000
DF = 128
H = 20
C = 16
W = 128

NC, NS = 2, 16
NW = NC * NS
PAD_N = 10240
CHUNK = 128
NCHR = E // CHUNK
CPB = NCHR // NW
XTR = NCHR - NW * CPB
RPT = PAD_N // NS

_mesh = lambda: plsc.VectorSubcoreMesh(core_axis_name="c", subcore_axis_name="s")
_sc_params = lambda: pltpu.CompilerParams(use_tc_tiling_on_sc=False, skip_device_barrier=True)


@functools.lru_cache(maxsize=None)
def _deg_kernel():
    @functools.partial(
        pl.kernel, mesh=_mesh(), compiler_params=_sc_params(),
        out_type=jax.ShapeDtypeStruct((NC, PAD_N), jnp.float32),
        scratch_types=[
            pltpu.VMEM((CPB, CHUNK), jnp.int32),
            pltpu.VMEM((1, CHUNK), jnp.int32),
            pltpu.VMEM((CHUNK,), jnp.float32),
            pltpu.VMEM_SHARED((PAD_N,), jnp.float32),
            pltpu.SemaphoreType.DMA,
        ],
    )
    def k(ei_hbm, zeros_hbm, out_hbm, dst_v, dst_x, ones_v, deg_sh, sem):
        c = lax.axis_index("c")
        s = lax.axis_index("s")
        wid = c * NS + s
        for i in range(CHUNK // 16):
            ones_v[pl.ds(i * 16, 16)] = jnp.ones((16,), jnp.float32)
        r0 = s * RPT
        pltpu.sync_copy(zeros_hbm.at[pl.ds(r0, RPT)], deg_sh.at[pl.ds(r0, RPT)])
        pltpu.sync_copy(ei_hbm.at[1].at[pl.ds(wid * CPB, CPB)], dst_v)

        @pl.when(wid < XTR)
        def _():
            pltpu.sync_copy(ei_hbm.at[1].at[pl.ds(NW * CPB + wid, 1)], dst_x)

        plsc.subcore_barrier()

        GRP = 13

        def body(i, carry):
            j = i * GRP
            for t in range(GRP):
                pltpu.async_copy(ones_v, deg_sh.at[dst_v.at[j + t]], sem,
                                 add=True)
            for t in range(GRP):
                pltpu.make_async_copy(ones_v, deg_sh.at[dst_v.at[j + t]],
                                      sem).wait()
            return carry

        lax.fori_loop(0, CPB // GRP, body, 0)

        @pl.when(wid < XTR)
        def _():
            pltpu.sync_copy(ones_v, deg_sh.at[dst_x.at[0]], add=True)

        plsc.subcore_barrier()
        pltpu.sync_copy(deg_sh.at[pl.ds(r0, RPT)], out_hbm.at[c].at[pl.ds(r0, RPT)])

    return k


@functools.lru_cache(maxsize=None)
def _agg_kernel(ws):
    @functools.partial(
        pl.kernel, mesh=_mesh(), compiler_params=_sc_params(),
        out_type=jax.ShapeDtypeStruct((NC, PAD_N, W), jnp.float32),
        scratch_types=[
            pltpu.VMEM((CPB, CHUNK), jnp.int32),
            pltpu.VMEM((CPB, CHUNK), jnp.int32),
            pltpu.VMEM((1, CHUNK), jnp.int32),
            pltpu.VMEM((1, CHUNK), jnp.int32),
            pltpu.VMEM((CHUNK, ws), jnp.float32),
            pltpu.VMEM((CHUNK, ws), jnp.float32),
            pltpu.VMEM((CHUNK, ws), jnp.float32),
            pltpu.VMEM((CHUNK, ws), jnp.float32),
            pltpu.VMEM_SHARED((PAD_N, ws), jnp.float32),
            pltpu.VMEM_SHARED((PAD_N, ws), jnp.float32),
            pltpu.SemaphoreType.DMA,
            pltpu.SemaphoreType.DMA,
            pltpu.SemaphoreType.DMA,
            pltpu.SemaphoreType.DMA,
            pltpu.SemaphoreType.DMA,
            pltpu.SemaphoreType.DMA,
            pltpu.SemaphoreType.DMA,
            pltpu.SemaphoreType.DMA,
        ],
    )
    def k(hp_hbm, ei_hbm, zeros_hbm, out_hbm,
          src_v, dst_v, src_x, dst_x, rows0, rows1, rows2, rows3,
          hp_sh, agg_sh, g0, g1, g2, g3, s0, s1, s2, s3):
        c = lax.axis_index("c")
        s = lax.axis_index("s")
        wid = c * NS + s
        r0 = s * RPT
        rows = (rows0, rows1, rows2, rows3)
        gsem = (g0, g1, g2, g3)
        ssem = (s0, s1, s2, s3)
        pltpu.sync_copy(zeros_hbm.at[pl.ds(r0, RPT)], agg_sh.at[pl.ds(r0, RPT)])
        pltpu.sync_copy(hp_hbm.at[pl.ds(r0, RPT), pl.ds(0, ws)],
                        hp_sh.at[pl.ds(r0, RPT)])
        pltpu.sync_copy(ei_hbm.at[0].at[pl.ds(wid * CPB, CPB)], src_v)
        pltpu.sync_copy(ei_hbm.at[1].at[pl.ds(wid * CPB, CPB)], dst_v)

        @pl.when(wid < XTR)
        def _():
            pltpu.sync_copy(ei_hbm.at[0].at[pl.ds(NW * CPB + wid, 1)], src_x)
            pltpu.sync_copy(ei_hbm.at[1].at[pl.ds(NW * CPB + wid, 1)], dst_x)

        plsc.subcore_barrier()

        pltpu.async_copy(hp_sh.at[src_v.at[0]], rows2, g2)
        pltpu.async_copy(hp_sh.at[src_v.at[1]], rows3, g3)

        def body(i, carry):
            j = i * 4
            for b in range(4):
                cid = j + b

                @pl.when(cid < CPB)
                def _():
                    pre = cid + 2

                    @pl.when(pre < CPB)
                    def _():
                        @pl.when(cid >= 2)
                        def _():
                            pltpu.make_async_copy(
                                rows[b], agg_sh.at[dst_v.at[cid - 2]],
                                ssem[b]).wait()

                        pltpu.async_copy(hp_sh.at[src_v.at[pre]],
                                         rows[b], gsem[b])

                    pltpu.make_async_copy(hp_sh.at[src_v.at[cid]],
                                          rows[(b + 2) % 4],
                                          gsem[(b + 2) % 4]).wait()
                    pltpu.async_copy(rows[(b + 2) % 4],
                                     agg_sh.at[dst_v.at[cid]],
                                     ssem[(b + 2) % 4], add=True)
            return carry

        lax.fori_loop(0, (CPB + 3) // 4, body, 0)
        for t in range(CPB - 4, CPB):
            pltpu.make_async_copy(rows[(t + 2) % 4], agg_sh.at[dst_v.at[t]],
                                  ssem[(t + 2) % 4]).wait()

        @pl.when(wid < XTR)
        def _():
            pltpu.async_copy(hp_sh.at[src_x.at[0]], rows0, g0).wait()
            pltpu.sync_copy(rows0, agg_sh.at[dst_x.at[0]], add=True)

        plsc.subcore_barrier()
        pltpu.sync_copy(agg_sh.at[pl.ds(r0, RPT)],
                        out_hbm.at[c].at[pl.ds(r0, RPT), pl.ds(0, ws)])

    return k


def _tc_scale_matmul(x_p, W1, degt):
    BN = 2048

    def body(x_ref, w_ref, deg_ref, hp_ref, dinv_ref):
        deg = deg_ref[:, 0:1] + deg_ref[:, 1:2] + 1.0
        dinv = lax.rsqrt(deg)
        h = jnp.dot(x_ref[...], w_ref[...], preferred_element_type=jnp.float32)
        hp_ref[...] = jnp.pad(h * dinv, ((0, 0), (0, W - H)))
        dinv_ref[...] = dinv

    return pl.pallas_call(
        body,
        grid=(PAD_N // BN,),
        in_specs=[
            pl.BlockSpec((BN, DF), lambda i: (i, 0)),
            pl.BlockSpec((DF, H), lambda i: (0, 0)),
            pl.BlockSpec((BN, NC), lambda i: (i, 0)),
        ],
        out_specs=[
            pl.BlockSpec((BN, W), lambda i: (i, 0)),
            pl.BlockSpec((BN, 1), lambda i: (i, 0)),
        ],
        out_shape=[
            jax.ShapeDtypeStruct((PAD_N, W), jnp.float32),
            jax.ShapeDtypeStruct((PAD_N, 1), jnp.float32),
        ],
    )(x_p, W1, degt)


def _tc_mid(aggp, hp, dinv, b1, W2):
    BN = 2048

    def body(a_ref, hp_ref, dinv_ref, b1_ref, w2_ref, out_ref):
        p = (a_ref[0] + a_ref[1] + hp_ref[...])[:, :H]
        h1 = jnp.maximum(dinv_ref[...] * p + b1_ref[...], 0.0)
        h2 = jnp.dot(h1, w2_ref[...], preferred_element_type=jnp.float32)
        out_ref[...] = jnp.pad(h2 * dinv_ref[...], ((0, 0), (0, W - C)))

    return pl.pallas_call(
        body,
        grid=(PAD_N // BN,),
        in_specs=[
            pl.BlockSpec((NC, BN, W), lambda i: (0, i, 0)),
            pl.BlockSpec((BN, W), lambda i: (i, 0)),
            pl.BlockSpec((BN, 1), lambda i: (i, 0)),
            pl.BlockSpec((1, H), lambda i: (0, 0)),
            pl.BlockSpec((H, C), lambda i: (0, 0)),
        ],
        out_specs=pl.BlockSpec((BN, W), lambda i: (i, 0)),
        out_shape=jax.ShapeDtypeStruct((PAD_N, W), jnp.float32),
    )(aggp, hp, dinv, b1, W2)


def _tc_final(aggp, h2p, dinv, b2):
    BN = 2048

    def body(a_ref, hp_ref, dinv_ref, b2_ref, out_ref):
        q = (a_ref[0] + a_ref[1] + hp_ref[...])[:, :C]
        z = dinv_ref[...] * q + b2_ref[...]
        m = jnp.max(z, axis=1, keepdims=True)
        e = jnp.exp(z - m)
        out_ref[...] = (z - m) - jnp.log(jnp.sum(e, axis=1, keepdims=True))

    return pl.pallas_call(
        body,
        grid=(PAD_N // BN,),
        in_specs=[
            pl.BlockSpec((NC, BN, W), lambda i: (0, i, 0)),
            pl.BlockSpec((BN, W), lambda i: (i, 0)),
            pl.BlockSpec((BN, 1), lambda i: (i, 0)),
            pl.BlockSpec((1, C), lambda i: (0, 0)),
        ],
        out_specs=pl.BlockSpec((BN, C), lambda i: (i, 0)),
        out_shape=jax.ShapeDtypeStruct((N, C), jnp.float32),
    )(aggp, h2p, dinv, b2)


def kernel(x, edge_index, W1, b1, W2, b2):
    ei3 = edge_index.astype(jnp.int32).reshape(2, NCHR, CHUNK)
    z1 = jnp.zeros((PAD_N,), jnp.float32)
    za = jnp.zeros((PAD_N, 24), jnp.float32)
    zb = jnp.zeros((PAD_N, 16), jnp.float32)

    degp = _deg_kernel()(ei3, z1)
    degt = degp.T
    hp, dinv = _tc_scale_matmul(x, W1, degt)
    agg1 = _agg_kernel(24)(hp, ei3, za)
    h2p = _tc_mid(agg1, hp, dinv, b1.reshape(1, H), W2)
    agg2 = _agg_kernel(16)(h2p, ei3, zb)
    return _tc_final(agg2, h2p, dinv, b2.reshape(1, C))
